# Initial kernel scaffold; baseline (speedup 1.0000x reference)
import jax, jax.numpy as jnp

B = 1024
D = 512
H = 8
KNN = 16
M = 100000

HI = jax.lax.Precision.HIGHEST


def kernel(x, mem_keys, mem_vals, W_img, b_img, Wq, bq, Wk, bk, Wv, bv,
           W_in, b_in, Wo, bo, W1, b1, W2, b2, W3, b3):
    Q = jnp.dot(x, W_img.T, precision=HI) + b_img
    q_sq = jnp.sum(Q * Q, axis=1, keepdims=True)
    k_sq = jnp.sum(mem_keys * mem_keys, axis=1)
    dist2 = q_sq - 2.0 * jnp.dot(Q, mem_keys.T, precision=HI) + k_sq[None, :]
    _, idx = jax.lax.top_k(-dist2, KNN)
    Kn = mem_keys[idx]
    Vn = mem_vals[idx]
    Qe = Q[:, None, :]
    q1 = Qe @ Wq.T + bq
    k1 = Kn @ Wk.T + bk
    v1 = Vn @ Wv.T + bv
    Wqi, Wki, Wvi = jnp.split(W_in, 3, axis=0)
    bqi, bki, bvi = jnp.split(b_in, 3, axis=0)
    q2 = q1 @ Wqi.T + bqi
    k2 = k1 @ Wki.T + bki
    v2 = v1 @ Wvi.T + bvi
    dh = D // H
    q2 = q2.reshape(B, 1, H, dh).transpose(0, 2, 1, 3)
    k2 = k2.reshape(B, KNN, H, dh).transpose(0, 2, 1, 3)
    v2 = v2.reshape(B, KNN, H, dh).transpose(0, 2, 1, 3)
    scale = 1.0 / jnp.sqrt(jnp.float32(dh))
    attn = jax.nn.softmax(jnp.einsum('bhqd,bhkd->bhqk', q2, k2) * scale, axis=-1)
    ctx = jnp.einsum('bhqk,bhkd->bhqd', attn, v2)
    ctx = ctx.transpose(0, 2, 1, 3).reshape(B, 1, D)
    prior = ctx @ Wo.T + bo
    feat1 = (Qe @ W1.T + b1)[:, 0, :]
    feat2 = (prior @ W2.T + b2)[:, 0, :]
    feat = jnp.concatenate([feat1, feat2], axis=1)
    feat = jax.nn.gelu(feat, approximate=False)
    feat = feat @ W3.T + b3
    return feat, Q


# speedup vs baseline: 5.5303x; 5.5303x over previous
import functools
import math

import jax
import jax.numpy as jnp
from jax import lax
from jax.experimental import pallas as pl
from jax.experimental.pallas import tpu as pltpu
from jax.experimental.pallas import tpu_sc as plsc

KNN = 16
H = 8
CW = 128
TK = 1024
NEG = -1e30

SC_NC = 2
SC_NS = 16
SC_NW = SC_NC * SC_NS

DIST_PREC = lax.Precision.DEFAULT
TAIL_PREC = lax.Precision.DEFAULT


def _dot_t(a, b, prec):
    return lax.dot_general(a, b, (((1,), (1,)), ((), ())), precision=prec,
                           preferred_element_type=jnp.float32)


def _dot(a, b, prec):
    return lax.dot_general(a, b, (((1,), (0,)), ((), ())), precision=prec,
                           preferred_element_type=jnp.float32)


def _proj_body(x_ref, w_ref, b_ref, q_ref):
    q_ref[...] = _dot_t(x_ref[...], w_ref[...], DIST_PREC) + b_ref[...]


def _project(x, W_img, b_img):
    B, D = x.shape
    return pl.pallas_call(
        _proj_body,
        out_shape=jax.ShapeDtypeStruct((B, D), jnp.float32),
    )(x, W_img, b_img.reshape(1, D))


def _dist_body(M, q_ref, k_ref, s_ref, tm_ref):
    j = pl.program_id(0)
    Qb = q_ref[...]
    Kb = k_ref[...]
    B = Qb.shape[0]
    qsq = jnp.sum(Qb * Qb, axis=1, keepdims=True)
    ksq = jnp.sum(Kb * Kb, axis=1)
    qk = _dot_t(Qb, Kb, DIST_PREC)
    dist2 = qsq - 2.0 * qk + ksq[None, :]
    col = lax.broadcasted_iota(jnp.int32, (B, TK), 1) + j * TK
    s = jnp.where(col < M, -dist2, NEG)
    s_ref[...] = s
    tm_ref[0] = jnp.max(s.reshape(B, TK // CW, CW), axis=2)


def _distances(Q, mem_keys):
    B, D = Q.shape
    M = mem_keys.shape[0]
    nstep = math.ceil(M / TK)
    nt = nstep * (TK // CW)
    S, TM3 = pl.pallas_call(
        functools.partial(_dist_body, M),
        grid=(nstep,),
        in_specs=[
            pl.BlockSpec((B, D), lambda j: (0, 0)),
            pl.BlockSpec((TK, D), lambda j: (j, 0)),
        ],
        out_specs=[
            pl.BlockSpec((B, TK), lambda j: (0, j)),
            pl.BlockSpec((1, B, TK // CW), lambda j: (j, 0, 0)),
        ],
        out_shape=[
            jax.ShapeDtypeStruct((B, nstep * TK), jnp.float32),
            jax.ShapeDtypeStruct((nstep, B, TK // CW), jnp.float32),
        ],
    )(Q, mem_keys)
    return S, TM3, nt


def _extract16(vals, ids):
    out_i = []
    big = jnp.int32(2**31 - 1)
    for _ in range(KNN):
        m = jnp.max(vals, axis=1, keepdims=True)
        sel = jnp.where(vals == m, ids, big)
        win = jnp.min(sel, axis=1, keepdims=True)
        out_i.append(win)
        vals = jnp.where(ids == win, -jnp.inf, vals)
    return jnp.concatenate(out_i, axis=1)


def _chunksel_body(nt, rc, tm_ref, gid_ref):
    i = pl.program_id(0)
    vals = tm_ref[...]
    cid = lax.broadcasted_iota(jnp.int32, vals.shape, 1)
    ti = _extract16(vals, cid)
    row = lax.broadcasted_iota(jnp.int32, ti.shape, 0) + i * rc
    gid_ref[...] = row * nt + ti


def _chunk_select(TM, nt):
    B = TM.shape[0]
    rc = min(256, B)
    return pl.pallas_call(
        functools.partial(_chunksel_body, nt, rc),
        grid=(B // rc,),
        in_specs=[pl.BlockSpec((rc, nt), lambda i: (i, 0))],
        out_specs=pl.BlockSpec((rc, KNN), lambda i: (i, 0)),
        out_shape=jax.ShapeDtypeStruct((B, KNN), jnp.int32),
    )(TM)


def _finalsel_body(nt, rc, cv_ref, gid_ref, idx_ref):
    i = pl.program_id(0)
    vals = cv_ref[...]
    gid = gid_ref[...]
    row = lax.broadcasted_iota(jnp.int32, gid.shape, 0) + i * rc
    base = (gid - row * nt) * CW
    lane = lax.broadcasted_iota(jnp.int32, (rc, CW), 1)
    ids = jnp.concatenate(
        [base[:, j:j + 1] + lane for j in range(KNN)], axis=1)
    idx_ref[...] = _extract16(vals, ids)


def _final_select(cand, GID, nt):
    B = GID.shape[0]
    rc = min(256, B)
    return pl.pallas_call(
        functools.partial(_finalsel_body, nt, rc),
        grid=(B // rc,),
        in_specs=[
            pl.BlockSpec((rc, KNN * CW), lambda i: (i, 0)),
            pl.BlockSpec((rc, KNN), lambda i: (i, 0)),
        ],
        out_specs=pl.BlockSpec((rc, KNN), lambda i: (i, 0)),
        out_shape=jax.ShapeDtypeStruct((B, KNN), jnp.int32),
    )(cand, GID)


def _sc_gather_chunks(S2, gid_flat):
    nrow = gid_flat.shape[0]
    per_w = nrow // SC_NW
    ch = 128
    mesh = plsc.VectorSubcoreMesh(core_axis_name="c", subcore_axis_name="s")

    @functools.partial(
        pl.kernel, mesh=mesh,
        out_type=jax.ShapeDtypeStruct((nrow, CW), jnp.float32),
        scratch_types=[
            pltpu.VMEM((ch,), jnp.int32),
            pltpu.VMEM((ch, CW), jnp.float32),
            pltpu.SemaphoreType.DMA,
        ],
    )
    def k(tbl, idxh, outh, idx_v, rows_v, sem):
        wid = lax.axis_index("s") * SC_NC + lax.axis_index("c")
        base = wid * per_w
        for c in range(per_w // ch):
            pltpu.sync_copy(idxh.at[pl.ds(base + c * ch, ch)], idx_v)
            pltpu.async_copy(tbl.at[idx_v], rows_v, sem).wait()
            pltpu.sync_copy(rows_v, outh.at[pl.ds(base + c * ch, ch)])

    return k(S2, gid_flat)


def _sc_gather_rows(mem_keys, mem_vals, idx_flat):
    nrow = idx_flat.shape[0]
    D = mem_keys.shape[1]
    per_w = nrow // SC_NW
    ch = 64
    mesh = plsc.VectorSubcoreMesh(core_axis_name="c", subcore_axis_name="s")

    @functools.partial(
        pl.kernel, mesh=mesh,
        out_type=(jax.ShapeDtypeStruct((nrow, D), jnp.float32),
                  jax.ShapeDtypeStruct((nrow, D), jnp.float32)),
        scratch_types=[
            pltpu.VMEM((ch,), jnp.int32),
            pltpu.VMEM((ch, D), jnp.float32),
            pltpu.VMEM((ch, D), jnp.float32),
            pltpu.SemaphoreType.DMA,
            pltpu.SemaphoreType.DMA,
        ],
    )
    def k(keys_h, vals_h, idxh, kout, vout, idx_v, krows, vrows, ksem, vsem):
        wid = lax.axis_index("s") * SC_NC + lax.axis_index("c")
        base = wid * per_w
        for c in range(per_w // ch):
            pltpu.sync_copy(idxh.at[pl.ds(base + c * ch, ch)], idx_v)
            kcp = pltpu.async_copy(keys_h.at[idx_v], krows, ksem)
            vcp = pltpu.async_copy(vals_h.at[idx_v], vrows, vsem)
            kcp.wait()
            pltpu.sync_copy(krows, kout.at[pl.ds(base + c * ch, ch)])
            vcp.wait()
            pltpu.sync_copy(vrows, vout.at[pl.ds(base + c * ch, ch)])

    return k(mem_keys, mem_vals, idx_flat)


def _tail_body(rt, q_ref, kn_ref, vn_ref, wq_ref, wk_ref, wv_ref, win_ref,
               bpre_ref, bin_ref, wo_ref, bo_ref, w1_ref, b1_ref, w2_ref,
               b2_ref, w3_ref, b3_ref, out_ref):
    D = q_ref.shape[1]
    dh = D // H
    Qb = q_ref[...]

    wiq = win_ref[0:D, :]
    wik = win_ref[D:2 * D, :]
    wiv = win_ref[2 * D:3 * D, :]
    Wqc = _dot(wiq, wq_ref[...], TAIL_PREC)
    Wkc = _dot(wik, wk_ref[...], TAIL_PREC)
    Wvc = _dot(wiv, wv_ref[...], TAIL_PREC)
    bqc = _dot_t(bpre_ref[0:1, :], wiq, TAIL_PREC) + bin_ref[0:1, :]
    bkc = _dot_t(bpre_ref[1:2, :], wik, TAIL_PREC) + bin_ref[1:2, :]
    bvc = _dot_t(bpre_ref[2:3, :], wiv, TAIL_PREC) + bin_ref[2:3, :]

    q2 = _dot_t(Qb, Wqc, TAIL_PREC) + bqc
    k2 = (_dot_t(kn_ref[...].reshape(KNN * rt, D), Wkc, TAIL_PREC)
          + bkc).reshape(KNN, rt, D)
    v2 = (_dot_t(vn_ref[...].reshape(KNN * rt, D), Wvc, TAIL_PREC)
          + bvc).reshape(KNN, rt, D)

    prod = k2 * q2[None, :, :]
    logits = jnp.concatenate(
        [jnp.sum(prod[:, :, h * dh:(h + 1) * dh], axis=2, keepdims=True)
         for h in range(H)], axis=2)
    logits = logits * (1.0 / math.sqrt(dh))
    mx = jnp.max(logits, axis=0, keepdims=True)
    p = jnp.exp(logits - mx)
    p = p / jnp.sum(p, axis=0, keepdims=True)
    pb = jnp.concatenate(
        [jnp.broadcast_to(p[:, :, h:h + 1], (KNN, rt, dh)) for h in range(H)],
        axis=2)
    ctx = jnp.sum(pb * v2, axis=0)

    prior = _dot_t(ctx, wo_ref[...], TAIL_PREC) + bo_ref[...]
    feat1 = _dot_t(Qb, w1_ref[...], TAIL_PREC) + b1_ref[...]
    feat2 = _dot_t(prior, w2_ref[...], TAIL_PREC) + b2_ref[...]
    feat = jnp.concatenate([feat1, feat2], axis=1)
    feat = feat * 0.5 * (1.0 + lax.erf(feat * (1.0 / math.sqrt(2.0))))
    out_ref[...] = _dot_t(feat, w3_ref[...], TAIL_PREC) + b3_ref[...]


def _tail(Q, KnT, VnT, Wq, Wk, Wv, W_in, b_pre, b_in3, Wo, bo, W1, b1,
          W2, b2, W3, b3):
    B, D = Q.shape
    rt = min(128, B)
    o1 = W1.shape[0]
    o2 = W2.shape[0]
    o3 = W3.shape[0]
    full = lambda a: pl.BlockSpec(a.shape, lambda i: (0,) * a.ndim)
    return pl.pallas_call(
        functools.partial(_tail_body, rt),
        grid=(B // rt,),
        in_specs=[
            pl.BlockSpec((rt, D), lambda i: (i, 0)),
            pl.BlockSpec((KNN, rt, D), lambda i: (0, i, 0)),
            pl.BlockSpec((KNN, rt, D), lambda i: (0, i, 0)),
            full(Wq), full(Wk), full(Wv), full(W_in), full(b_pre),
            full(b_in3), full(Wo), full(bo), full(W1), full(b1),
            full(W2), full(b2), full(W3), full(b3),
        ],
        out_specs=pl.BlockSpec((rt, o3), lambda i: (i, 0)),
        out_shape=jax.ShapeDtypeStruct((B, o3), jnp.float32),
    )(Q, KnT, VnT, Wq, Wk, Wv, W_in, b_pre, b_in3, Wo, bo, W1, b1,
      W2, b2, W3, b3)


def kernel(x, mem_keys, mem_vals, W_img, b_img, Wq, bq, Wk, bk, Wv, bv,
           W_in, b_in, Wo, bo, W1, b1, W2, b2, W3, b3):
    B, D = x.shape
    M = mem_keys.shape[0]

    Q = _project(x, W_img, b_img)
    S, TM3, nt = _distances(Q, mem_keys)
    TM = TM3.transpose(1, 0, 2).reshape(B, nt)
    GID = _chunk_select(TM, nt)
    cand = _sc_gather_chunks(S.reshape(B * nt, CW), GID.reshape(-1))
    idx = _final_select(cand.reshape(B, KNN * CW), GID, nt)

    idx_t = idx.T.reshape(-1)
    KnF, VnF = _sc_gather_rows(mem_keys, mem_vals, idx_t)
    KnT = KnF.reshape(KNN, B, D)
    VnT = VnF.reshape(KNN, B, D)

    b_pre = jnp.stack([bq, bk, bv], axis=0)
    b_in3 = b_in.reshape(3, D)
    feat = _tail(Q, KnT, VnT, Wq, Wk, Wv, W_in, b_pre, b_in3, Wo,
                 bo.reshape(1, D), W1, b1.reshape(1, -1), W2,
                 b2.reshape(1, -1), W3, b3.reshape(1, -1))
    return feat, Q


# speedup vs baseline: 7.5908x; 1.3726x over previous
import functools
import math

import jax
import jax.numpy as jnp
from jax import lax
from jax.experimental import pallas as pl
from jax.experimental.pallas import tpu as pltpu
from jax.experimental.pallas import tpu_sc as plsc

KNN = 16
H = 8
CW = 128
TK = 1024
NEG = -1e30

SC_NC = 2
SC_NS = 16
SC_NW = SC_NC * SC_NS

DIST_PREC = lax.Precision.DEFAULT
TAIL_PREC = lax.Precision.DEFAULT


def _dot_t(a, b, prec):
    return lax.dot_general(a, b, (((1,), (1,)), ((), ())), precision=prec,
                           preferred_element_type=jnp.float32)


def _dot(a, b, prec):
    return lax.dot_general(a, b, (((1,), (0,)), ((), ())), precision=prec,
                           preferred_element_type=jnp.float32)


def _proj_body(x_ref, w_ref, b_ref, q_ref):
    q_ref[...] = _dot_t(x_ref[...], w_ref[...], DIST_PREC) + b_ref[...]


def _project(x, W_img, b_img):
    B, D = x.shape
    return pl.pallas_call(
        _proj_body,
        out_shape=jax.ShapeDtypeStruct((B, D), jnp.float32),
    )(x, W_img, b_img.reshape(1, D))


def _dist_body(M, q_ref, k_ref, s_ref, tm_ref):
    j = pl.program_id(0)
    Qb = q_ref[...]
    Kb = k_ref[...]
    B = Qb.shape[0]
    qsq = jnp.sum(Qb * Qb, axis=1, keepdims=True)
    ksq = jnp.sum(Kb * Kb, axis=1)
    qk = _dot_t(Qb, Kb, DIST_PREC)
    dist2 = qsq - 2.0 * qk + ksq[None, :]
    col = lax.broadcasted_iota(jnp.int32, (B, TK), 1) + j * TK
    s = jnp.where(col < M, -dist2, NEG)
    s_ref[...] = s.reshape(B, TK // CW, CW)
    tm_ref[0] = jnp.max(s.reshape(B, TK // CW, CW), axis=2)


def _distances(Q, mem_keys):
    B, D = Q.shape
    M = mem_keys.shape[0]
    nstep = math.ceil(M / TK)
    nt = nstep * (TK // CW)
    S, TM3 = pl.pallas_call(
        functools.partial(_dist_body, M),
        grid=(nstep,),
        in_specs=[
            pl.BlockSpec((B, D), lambda j: (0, 0)),
            pl.BlockSpec((TK, D), lambda j: (j, 0)),
        ],
        out_specs=[
            pl.BlockSpec((B, TK // CW, CW), lambda j: (0, j, 0)),
            pl.BlockSpec((1, B, TK // CW), lambda j: (j, 0, 0)),
        ],
        out_shape=[
            jax.ShapeDtypeStruct((B, nt, CW), jnp.float32),
            jax.ShapeDtypeStruct((nstep, B, TK // CW), jnp.float32),
        ],
    )(Q, mem_keys)
    return S, TM3, nt


def _extract16(vals, ids):
    out_i = []
    big = jnp.int32(2**31 - 1)
    for _ in range(KNN):
        m = jnp.max(vals, axis=1, keepdims=True)
        sel = jnp.where(vals == m, ids, big)
        win = jnp.min(sel, axis=1, keepdims=True)
        out_i.append(win)
        vals = jnp.where(ids == win, -jnp.inf, vals)
    return jnp.concatenate(out_i, axis=1)


def _chunksel_body(nt, rc, tm_ref, gid_ref):
    i = pl.program_id(0)
    vals = tm_ref[...]
    cid = lax.broadcasted_iota(jnp.int32, vals.shape, 1)
    ti = _extract16(vals, cid)
    row = lax.broadcasted_iota(jnp.int32, ti.shape, 0) + i * rc
    gid_ref[...] = row * nt + ti


def _chunk_select(TM, nt):
    B = TM.shape[0]
    rc = min(256, B)
    return pl.pallas_call(
        functools.partial(_chunksel_body, nt, rc),
        grid=(B // rc,),
        in_specs=[pl.BlockSpec((rc, nt), lambda i: (i, 0))],
        out_specs=pl.BlockSpec((rc, KNN), lambda i: (i, 0)),
        out_shape=jax.ShapeDtypeStruct((B, KNN), jnp.int32),
    )(TM)


def _finalsel_body(nt, rc, cv_ref, gid_ref, idx_ref):
    i = pl.program_id(0)
    vals = cv_ref[...]
    gid = gid_ref[...]
    row = lax.broadcasted_iota(jnp.int32, gid.shape, 0) + i * rc
    base = (gid - row * nt) * CW
    lane = lax.broadcasted_iota(jnp.int32, (rc, CW), 1)
    ids = jnp.concatenate(
        [base[:, j:j + 1] + lane for j in range(KNN)], axis=1)
    idx_ref[...] = _extract16(vals, ids)


def _final_select(cand, GID, nt):
    B = GID.shape[0]
    rc = min(256, B)
    return pl.pallas_call(
        functools.partial(_finalsel_body, nt, rc),
        grid=(B // rc,),
        in_specs=[
            pl.BlockSpec((rc, KNN * CW), lambda i: (i, 0)),
            pl.BlockSpec((rc, KNN), lambda i: (i, 0)),
        ],
        out_specs=pl.BlockSpec((rc, KNN), lambda i: (i, 0)),
        out_shape=jax.ShapeDtypeStruct((B, KNN), jnp.int32),
    )(cand, GID)


def _sc_gather_chunks(S2, gid_flat):
    nrow = gid_flat.shape[0]
    per_w = nrow // SC_NW
    ch = 128
    mesh = plsc.VectorSubcoreMesh(core_axis_name="c", subcore_axis_name="s")

    @functools.partial(
        pl.kernel, mesh=mesh,
        out_type=jax.ShapeDtypeStruct((nrow, CW), jnp.float32),
        scratch_types=[
            pltpu.VMEM((ch,), jnp.int32),
            pltpu.VMEM((ch, CW), jnp.float32),
            pltpu.SemaphoreType.DMA,
        ],
    )
    def k(tbl, idxh, outh, idx_v, rows_v, sem):
        wid = lax.axis_index("s") * SC_NC + lax.axis_index("c")
        base = wid * per_w
        for c in range(per_w // ch):
            pltpu.sync_copy(idxh.at[pl.ds(base + c * ch, ch)], idx_v)
            pltpu.async_copy(tbl.at[idx_v], rows_v, sem).wait()
            pltpu.sync_copy(rows_v, outh.at[pl.ds(base + c * ch, ch)])

    return k(S2, gid_flat)


def _sc_gather_rows(mem_keys, mem_vals, idx_flat):
    nrow = idx_flat.shape[0]
    D = mem_keys.shape[1]
    per_w = nrow // SC_NW
    ch = 64
    mesh = plsc.VectorSubcoreMesh(core_axis_name="c", subcore_axis_name="s")

    @functools.partial(
        pl.kernel, mesh=mesh,
        out_type=(jax.ShapeDtypeStruct((nrow, D), jnp.float32),
                  jax.ShapeDtypeStruct((nrow, D), jnp.float32)),
        scratch_types=[
            pltpu.VMEM((ch,), jnp.int32),
            pltpu.VMEM((ch, D), jnp.float32),
            pltpu.VMEM((ch, D), jnp.float32),
            pltpu.SemaphoreType.DMA,
            pltpu.SemaphoreType.DMA,
        ],
    )
    def k(keys_h, vals_h, idxh, kout, vout, idx_v, krows, vrows, ksem, vsem):
        wid = lax.axis_index("s") * SC_NC + lax.axis_index("c")
        base = wid * per_w
        for c in range(per_w // ch):
            pltpu.sync_copy(idxh.at[pl.ds(base + c * ch, ch)], idx_v)
            kcp = pltpu.async_copy(keys_h.at[idx_v], krows, ksem)
            vcp = pltpu.async_copy(vals_h.at[idx_v], vrows, vsem)
            kcp.wait()
            pltpu.sync_copy(krows, kout.at[pl.ds(base + c * ch, ch)])
            vcp.wait()
            pltpu.sync_copy(vrows, vout.at[pl.ds(base + c * ch, ch)])

    return k(mem_keys, mem_vals, idx_flat)


def _tail_body(rt, q_ref, kn_ref, vn_ref, wq_ref, wk_ref, wv_ref, win_ref,
               bpre_ref, bin_ref, wo_ref, bo_ref, w1_ref, b1_ref, w2_ref,
               b2_ref, w3_ref, b3_ref, out_ref):
    D = q_ref.shape[1]
    dh = D // H
    Qb = q_ref[...]

    wiq = win_ref[0:D, :]
    wik = win_ref[D:2 * D, :]
    wiv = win_ref[2 * D:3 * D, :]
    Wqc = _dot(wiq, wq_ref[...], TAIL_PREC)
    Wkc = _dot(wik, wk_ref[...], TAIL_PREC)
    Wvc = _dot(wiv, wv_ref[...], TAIL_PREC)
    bqc = _dot_t(bpre_ref[0:1, :], wiq, TAIL_PREC) + bin_ref[0:1, :]
    bkc = _dot_t(bpre_ref[1:2, :], wik, TAIL_PREC) + bin_ref[1:2, :]
    bvc = _dot_t(bpre_ref[2:3, :], wiv, TAIL_PREC) + bin_ref[2:3, :]

    q2 = _dot_t(Qb, Wqc, TAIL_PREC) + bqc
    k2 = (_dot_t(kn_ref[...].reshape(KNN * rt, D), Wkc, TAIL_PREC)
          + bkc).reshape(KNN, rt, D)
    v2 = (_dot_t(vn_ref[...].reshape(KNN * rt, D), Wvc, TAIL_PREC)
          + bvc).reshape(KNN, rt, D)

    prod = k2 * q2[None, :, :]
    logits = jnp.concatenate(
        [jnp.sum(prod[:, :, h * dh:(h + 1) * dh], axis=2, keepdims=True)
         for h in range(H)], axis=2)
    logits = logits * (1.0 / math.sqrt(dh))
    mx = jnp.max(logits, axis=0, keepdims=True)
    p = jnp.exp(logits - mx)
    p = p / jnp.sum(p, axis=0, keepdims=True)
    pb = jnp.concatenate(
        [jnp.broadcast_to(p[:, :, h:h + 1], (KNN, rt, dh)) for h in range(H)],
        axis=2)
    ctx = jnp.sum(pb * v2, axis=0)

    prior = _dot_t(ctx, wo_ref[...], TAIL_PREC) + bo_ref[...]
    feat1 = _dot_t(Qb, w1_ref[...], TAIL_PREC) + b1_ref[...]
    feat2 = _dot_t(prior, w2_ref[...], TAIL_PREC) + b2_ref[...]
    feat = jnp.concatenate([feat1, feat2], axis=1)
    feat = feat * 0.5 * (1.0 + lax.erf(feat * (1.0 / math.sqrt(2.0))))
    out_ref[...] = _dot_t(feat, w3_ref[...], TAIL_PREC) + b3_ref[...]


def _tail(Q, KnT, VnT, Wq, Wk, Wv, W_in, b_pre, b_in3, Wo, bo, W1, b1,
          W2, b2, W3, b3):
    B, D = Q.shape
    rt = min(128, B)
    o1 = W1.shape[0]
    o2 = W2.shape[0]
    o3 = W3.shape[0]
    full = lambda a: pl.BlockSpec(a.shape, lambda i: (0,) * a.ndim)
    return pl.pallas_call(
        functools.partial(_tail_body, rt),
        grid=(B // rt,),
        in_specs=[
            pl.BlockSpec((rt, D), lambda i: (i, 0)),
            pl.BlockSpec((KNN, rt, D), lambda i: (0, i, 0)),
            pl.BlockSpec((KNN, rt, D), lambda i: (0, i, 0)),
            full(Wq), full(Wk), full(Wv), full(W_in), full(b_pre),
            full(b_in3), full(Wo), full(bo), full(W1), full(b1),
            full(W2), full(b2), full(W3), full(b3),
        ],
        out_specs=pl.BlockSpec((rt, o3), lambda i: (i, 0)),
        out_shape=jax.ShapeDtypeStruct((B, o3), jnp.float32),
    )(Q, KnT, VnT, Wq, Wk, Wv, W_in, b_pre, b_in3, Wo, bo, W1, b1,
      W2, b2, W3, b3)


def kernel(x, mem_keys, mem_vals, W_img, b_img, Wq, bq, Wk, bk, Wv, bv,
           W_in, b_in, Wo, bo, W1, b1, W2, b2, W3, b3):
    B, D = x.shape
    M = mem_keys.shape[0]

    Q = _project(x, W_img, b_img)
    S, TM3, nt = _distances(Q, mem_keys)
    TM = TM3.transpose(1, 0, 2).reshape(B, nt)
    GID = _chunk_select(TM, nt)
    cand = _sc_gather_chunks(S.reshape(B * nt, CW), GID.reshape(-1))
    idx = _final_select(cand.reshape(B, KNN * CW), GID, nt)

    idx_t = idx.T.reshape(-1)
    KnF, VnF = _sc_gather_rows(mem_keys, mem_vals, idx_t)
    KnT = KnF.reshape(KNN, B, D)
    VnT = VnF.reshape(KNN, B, D)

    b_pre = jnp.stack([bq, bk, bv], axis=0)
    b_in3 = b_in.reshape(3, D)
    feat = _tail(Q, KnT, VnT, Wq, Wk, Wv, W_in, b_pre, b_in3, Wo,
                 bo.reshape(1, D), W1, b1.reshape(1, -1), W2,
                 b2.reshape(1, -1), W3, b3.reshape(1, -1))
    return feat, Q


# speedup vs baseline: 7.6313x; 1.0053x over previous
import functools
import math

import jax
import jax.numpy as jnp
from jax import lax
from jax.experimental import pallas as pl
from jax.experimental.pallas import tpu as pltpu
from jax.experimental.pallas import tpu_sc as plsc

KNN = 16
H = 8
CW = 128
TK = 1024
NEG = -1e30

SC_NC = 2
SC_NS = 16
SC_NW = SC_NC * SC_NS

DIST_PREC = lax.Precision.DEFAULT
TAIL_PREC = lax.Precision.DEFAULT


def _dot_t(a, b, prec):
    return lax.dot_general(a, b, (((1,), (1,)), ((), ())), precision=prec,
                           preferred_element_type=jnp.float32)


def _dot(a, b, prec):
    return lax.dot_general(a, b, (((1,), (0,)), ((), ())), precision=prec,
                           preferred_element_type=jnp.float32)


def _proj_body(x_ref, w_ref, b_ref, wq_ref, wk_ref, wv_ref, win_ref,
               bpre_ref, bin_ref, q_ref, wqc_ref, wkc_ref, wvc_ref, bc_ref):
    D = x_ref.shape[1]
    q_ref[...] = _dot_t(x_ref[...], w_ref[...], DIST_PREC) + b_ref[...]
    wiq = win_ref[0:D, :]
    wik = win_ref[D:2 * D, :]
    wiv = win_ref[2 * D:3 * D, :]
    wqc_ref[...] = _dot(wiq, wq_ref[...], TAIL_PREC)
    wkc_ref[...] = _dot(wik, wk_ref[...], TAIL_PREC)
    wvc_ref[...] = _dot(wiv, wv_ref[...], TAIL_PREC)
    bc_ref[...] = jnp.concatenate(
        [_dot_t(bpre_ref[0:1, :], wiq, TAIL_PREC) + bin_ref[0:1, :],
         _dot_t(bpre_ref[1:2, :], wik, TAIL_PREC) + bin_ref[1:2, :],
         _dot_t(bpre_ref[2:3, :], wiv, TAIL_PREC) + bin_ref[2:3, :]], axis=0)


def _project(x, W_img, b_img, Wq, Wk, Wv, W_in, b_pre, b_in3):
    B, D = x.shape
    f32 = jnp.float32
    return pl.pallas_call(
        _proj_body,
        out_shape=[jax.ShapeDtypeStruct((B, D), f32),
                   jax.ShapeDtypeStruct((D, D), f32),
                   jax.ShapeDtypeStruct((D, D), f32),
                   jax.ShapeDtypeStruct((D, D), f32),
                   jax.ShapeDtypeStruct((3, D), f32)],
    )(x, W_img, b_img.reshape(1, D), Wq, Wk, Wv, W_in, b_pre, b_in3)


def _dist_body(M, q_ref, k_ref, s_ref, tm_ref):
    j = pl.program_id(0)
    Qb = q_ref[...]
    Kb = k_ref[...]
    B = Qb.shape[0]
    qsq = jnp.sum(Qb * Qb, axis=1, keepdims=True)
    ksq = jnp.sum(Kb * Kb, axis=1)
    qk = _dot_t(Qb, Kb, DIST_PREC)
    dist2 = qsq - 2.0 * qk + ksq[None, :]
    col = lax.broadcasted_iota(jnp.int32, (B, TK), 1) + j * TK
    s = jnp.where(col < M, -dist2, NEG)
    s_ref[...] = s.reshape(B, TK // CW, CW)
    tm_ref[0] = jnp.max(s.reshape(B, TK // CW, CW), axis=2)


def _distances(Q, mem_keys):
    B, D = Q.shape
    M = mem_keys.shape[0]
    nstep = math.ceil(M / TK)
    nt = nstep * (TK // CW)
    S, TM3 = pl.pallas_call(
        functools.partial(_dist_body, M),
        grid=(nstep,),
        in_specs=[
            pl.BlockSpec((B, D), lambda j: (0, 0)),
            pl.BlockSpec((TK, D), lambda j: (j, 0)),
        ],
        out_specs=[
            pl.BlockSpec((B, TK // CW, CW), lambda j: (0, j, 0)),
            pl.BlockSpec((1, B, TK // CW), lambda j: (j, 0, 0)),
        ],
        out_shape=[
            jax.ShapeDtypeStruct((B, nt, CW), jnp.float32),
            jax.ShapeDtypeStruct((nstep, B, TK // CW), jnp.float32),
        ],
    )(Q, mem_keys)
    return S, TM3, nt


def _extract16(vals, ids):
    out_i = []
    big = jnp.int32(2**31 - 1)
    for _ in range(KNN):
        m = jnp.max(vals, axis=1, keepdims=True)
        sel = jnp.where(vals == m, ids, big)
        win = jnp.min(sel, axis=1, keepdims=True)
        out_i.append(win)
        vals = jnp.where(ids == win, -jnp.inf, vals)
    return jnp.concatenate(out_i, axis=1)


def _chunksel_body(nt, rc, tm_ref, gid_ref):
    i = pl.program_id(0)
    vals = tm_ref[...]
    cid = lax.broadcasted_iota(jnp.int32, vals.shape, 1)
    ti = _extract16(vals, cid)
    row = lax.broadcasted_iota(jnp.int32, ti.shape, 0) + i * rc
    gid_ref[...] = row * nt + ti


def _chunk_select(TM, nt):
    B = TM.shape[0]
    rc = min(256, B)
    return pl.pallas_call(
        functools.partial(_chunksel_body, nt, rc),
        grid=(B // rc,),
        in_specs=[pl.BlockSpec((rc, nt), lambda i: (i, 0))],
        out_specs=pl.BlockSpec((rc, KNN), lambda i: (i, 0)),
        out_shape=jax.ShapeDtypeStruct((B, KNN), jnp.int32),
    )(TM)


def _finalsel_body(nt, rc, cv_ref, gid_ref, idx_ref):
    i = pl.program_id(0)
    vals = cv_ref[...]
    gid = gid_ref[...]
    row = lax.broadcasted_iota(jnp.int32, gid.shape, 0) + i * rc
    base = (gid - row * nt) * CW
    lane = lax.broadcasted_iota(jnp.int32, (rc, CW), 1)
    ids = jnp.concatenate(
        [base[:, j:j + 1] + lane for j in range(KNN)], axis=1)
    idx_ref[...] = _extract16(vals, ids)


def _final_select(cand, GID, nt):
    B = GID.shape[0]
    rc = min(256, B)
    return pl.pallas_call(
        functools.partial(_finalsel_body, nt, rc),
        grid=(B // rc,),
        in_specs=[
            pl.BlockSpec((rc, KNN * CW), lambda i: (i, 0)),
            pl.BlockSpec((rc, KNN), lambda i: (i, 0)),
        ],
        out_specs=pl.BlockSpec((rc, KNN), lambda i: (i, 0)),
        out_shape=jax.ShapeDtypeStruct((B, KNN), jnp.int32),
    )(cand, GID)


def _sc_gather_chunks(S2, gid_flat):
    nrow = gid_flat.shape[0]
    per_w = nrow // SC_NW
    ch = 128
    mesh = plsc.VectorSubcoreMesh(core_axis_name="c", subcore_axis_name="s")

    @functools.partial(
        pl.kernel, mesh=mesh,
        out_type=jax.ShapeDtypeStruct((nrow, CW), jnp.float32),
        scratch_types=[
            pltpu.VMEM((ch,), jnp.int32),
            pltpu.VMEM((ch, CW), jnp.float32),
            pltpu.SemaphoreType.DMA,
        ],
    )
    def k(tbl, idxh, outh, idx_v, rows_v, sem):
        wid = lax.axis_index("s") * SC_NC + lax.axis_index("c")
        base = wid * per_w
        for c in range(per_w // ch):
            pltpu.sync_copy(idxh.at[pl.ds(base + c * ch, ch)], idx_v)
            pltpu.async_copy(tbl.at[idx_v], rows_v, sem).wait()
            pltpu.sync_copy(rows_v, outh.at[pl.ds(base + c * ch, ch)])

    return k(S2, gid_flat)


def _sc_gather_rows(mem_keys, mem_vals, idx_flat):
    nrow = idx_flat.shape[0]
    D = mem_keys.shape[1]
    per_w = nrow // SC_NW
    ch = 64
    mesh = plsc.VectorSubcoreMesh(core_axis_name="c", subcore_axis_name="s")

    @functools.partial(
        pl.kernel, mesh=mesh,
        out_type=(jax.ShapeDtypeStruct((nrow, D), jnp.float32),
                  jax.ShapeDtypeStruct((nrow, D), jnp.float32)),
        scratch_types=[
            pltpu.VMEM((ch,), jnp.int32),
            pltpu.VMEM((ch, D), jnp.float32),
            pltpu.VMEM((ch, D), jnp.float32),
            pltpu.SemaphoreType.DMA,
            pltpu.SemaphoreType.DMA,
        ],
    )
    def k(keys_h, vals_h, idxh, kout, vout, idx_v, krows, vrows, ksem, vsem):
        wid = lax.axis_index("s") * SC_NC + lax.axis_index("c")
        base = wid * per_w
        for c in range(per_w // ch):
            pltpu.sync_copy(idxh.at[pl.ds(base + c * ch, ch)], idx_v)
            kcp = pltpu.async_copy(keys_h.at[idx_v], krows, ksem)
            vcp = pltpu.async_copy(vals_h.at[idx_v], vrows, vsem)
            kcp.wait()
            pltpu.sync_copy(krows, kout.at[pl.ds(base + c * ch, ch)])
            vcp.wait()
            pltpu.sync_copy(vrows, vout.at[pl.ds(base + c * ch, ch)])

    return k(mem_keys, mem_vals, idx_flat)


def _tail_body(rt, q_ref, kn_ref, vn_ref, wqc_ref, wkc_ref, wvc_ref,
               bc_ref, wo_ref, bo_ref, w1_ref, b1_ref, w2_ref,
               b2_ref, w3_ref, b3_ref, out_ref):
    D = q_ref.shape[1]
    dh = D // H
    Qb = q_ref[...]

    q2 = _dot_t(Qb, wqc_ref[...], TAIL_PREC) + bc_ref[0:1, :]
    k2 = (_dot_t(kn_ref[...].reshape(KNN * rt, D), wkc_ref[...], TAIL_PREC)
          + bc_ref[1:2, :]).reshape(KNN, rt, D)
    v2 = (_dot_t(vn_ref[...].reshape(KNN * rt, D), wvc_ref[...], TAIL_PREC)
          + bc_ref[2:3, :]).reshape(KNN, rt, D)

    prod = k2 * q2[None, :, :]
    logits = jnp.concatenate(
        [jnp.sum(prod[:, :, h * dh:(h + 1) * dh], axis=2, keepdims=True)
         for h in range(H)], axis=2)
    logits = logits * (1.0 / math.sqrt(dh))
    mx = jnp.max(logits, axis=0, keepdims=True)
    p = jnp.exp(logits - mx)
    p = p / jnp.sum(p, axis=0, keepdims=True)
    pb = jnp.concatenate(
        [jnp.broadcast_to(p[:, :, h:h + 1], (KNN, rt, dh)) for h in range(H)],
        axis=2)
    ctx = jnp.sum(pb * v2, axis=0)

    prior = _dot_t(ctx, wo_ref[...], TAIL_PREC) + bo_ref[...]
    feat1 = _dot_t(Qb, w1_ref[...], TAIL_PREC) + b1_ref[...]
    feat2 = _dot_t(prior, w2_ref[...], TAIL_PREC) + b2_ref[...]
    feat = jnp.concatenate([feat1, feat2], axis=1)
    feat = feat * 0.5 * (1.0 + lax.erf(feat * (1.0 / math.sqrt(2.0))))
    out_ref[...] = _dot_t(feat, w3_ref[...], TAIL_PREC) + b3_ref[...]


def _tail(Q, KnT, VnT, Wqc, Wkc, Wvc, bc, Wo, bo, W1, b1, W2, b2, W3, b3):
    B, D = Q.shape
    rt = min(128, B)
    o3 = W3.shape[0]
    full = lambda a: pl.BlockSpec(a.shape, lambda i: (0,) * a.ndim)
    return pl.pallas_call(
        functools.partial(_tail_body, rt),
        grid=(B // rt,),
        in_specs=[
            pl.BlockSpec((rt, D), lambda i: (i, 0)),
            pl.BlockSpec((KNN, rt, D), lambda i: (0, i, 0)),
            pl.BlockSpec((KNN, rt, D), lambda i: (0, i, 0)),
            full(Wqc), full(Wkc), full(Wvc), full(bc), full(Wo), full(bo),
            full(W1), full(b1), full(W2), full(b2), full(W3), full(b3),
        ],
        out_specs=pl.BlockSpec((rt, o3), lambda i: (i, 0)),
        out_shape=jax.ShapeDtypeStruct((B, o3), jnp.float32),
    )(Q, KnT, VnT, Wqc, Wkc, Wvc, bc, Wo, bo, W1, b1, W2, b2, W3, b3)


def kernel(x, mem_keys, mem_vals, W_img, b_img, Wq, bq, Wk, bk, Wv, bv,
           W_in, b_in, Wo, bo, W1, b1, W2, b2, W3, b3):
    B, D = x.shape
    M = mem_keys.shape[0]

    b_pre = jnp.stack([bq, bk, bv], axis=0)
    b_in3 = b_in.reshape(3, D)
    Q, Wqc, Wkc, Wvc, bc = _project(x, W_img, b_img, Wq, Wk, Wv, W_in,
                                    b_pre, b_in3)
    S, TM3, nt = _distances(Q, mem_keys)
    TM = TM3.transpose(1, 0, 2).reshape(B, nt)
    GID = _chunk_select(TM, nt)
    cand = _sc_gather_chunks(S.reshape(B * nt, CW), GID.reshape(-1))
    idx = _final_select(cand.reshape(B, KNN * CW), GID, nt)

    idx_t = idx.T.reshape(-1)
    KnF, VnF = _sc_gather_rows(mem_keys, mem_vals, idx_t)
    KnT = KnF.reshape(KNN, B, D)
    VnT = VnF.reshape(KNN, B, D)

    feat = _tail(Q, KnT, VnT, Wqc, Wkc, Wvc, bc, Wo,
                 bo.reshape(1, D), W1, b1.reshape(1, -1), W2,
                 b2.reshape(1, -1), W3, b3.reshape(1, -1))
    return feat, Q


# speedup vs baseline: 7.7903x; 1.0208x over previous
import functools
import math

import jax
import jax.numpy as jnp
from jax import lax
from jax.experimental import pallas as pl
from jax.experimental.pallas import tpu as pltpu
from jax.experimental.pallas import tpu_sc as plsc

KNN = 16
H = 8
CW = 128
TK = 2048
NEG = -1e30

SC_NC = 2
SC_NS = 16
SC_NW = SC_NC * SC_NS

DIST_PREC = lax.Precision.DEFAULT
TAIL_PREC = lax.Precision.DEFAULT


def _dot_t(a, b, prec):
    return lax.dot_general(a, b, (((1,), (1,)), ((), ())), precision=prec,
                           preferred_element_type=jnp.float32)


def _dot(a, b, prec):
    return lax.dot_general(a, b, (((1,), (0,)), ((), ())), precision=prec,
                           preferred_element_type=jnp.float32)


def _proj_body(x_ref, w_ref, b_ref, wq_ref, wk_ref, wv_ref, win_ref,
               bpre_ref, bin_ref, q_ref, wqc_ref, wkc_ref, wvc_ref, bc_ref):
    D = x_ref.shape[1]
    q_ref[...] = _dot_t(x_ref[...], w_ref[...], DIST_PREC) + b_ref[...]
    wiq = win_ref[0:D, :]
    wik = win_ref[D:2 * D, :]
    wiv = win_ref[2 * D:3 * D, :]
    wqc_ref[...] = _dot(wiq, wq_ref[...], TAIL_PREC)
    wkc_ref[...] = _dot(wik, wk_ref[...], TAIL_PREC)
    wvc_ref[...] = _dot(wiv, wv_ref[...], TAIL_PREC)
    bc_ref[...] = jnp.concatenate(
        [_dot_t(bpre_ref[0:1, :], wiq, TAIL_PREC) + bin_ref[0:1, :],
         _dot_t(bpre_ref[1:2, :], wik, TAIL_PREC) + bin_ref[1:2, :],
         _dot_t(bpre_ref[2:3, :], wiv, TAIL_PREC) + bin_ref[2:3, :]], axis=0)


def _project(x, W_img, b_img, Wq, Wk, Wv, W_in, b_pre, b_in3):
    B, D = x.shape
    f32 = jnp.float32
    return pl.pallas_call(
        _proj_body,
        out_shape=[jax.ShapeDtypeStruct((B, D), f32),
                   jax.ShapeDtypeStruct((D, D), f32),
                   jax.ShapeDtypeStruct((D, D), f32),
                   jax.ShapeDtypeStruct((D, D), f32),
                   jax.ShapeDtypeStruct((3, D), f32)],
    )(x, W_img, b_img.reshape(1, D), Wq, Wk, Wv, W_in, b_pre, b_in3)


def _dist_body(M, q_ref, k_ref, s_ref, tm_ref):
    j = pl.program_id(0)
    Qb = q_ref[...]
    Kb = k_ref[...]
    B = Qb.shape[0]
    qsq = jnp.sum(Qb * Qb, axis=1, keepdims=True)
    ksq = jnp.sum(Kb * Kb, axis=1)
    qk = _dot_t(Qb, Kb, DIST_PREC)
    dist2 = qsq - 2.0 * qk + ksq[None, :]
    col = lax.broadcasted_iota(jnp.int32, (B, TK), 1) + j * TK
    s = jnp.where(col < M, -dist2, NEG)
    s_ref[...] = s.reshape(B, TK // CW, CW)
    tm_ref[0] = jnp.max(s.reshape(B, TK // CW, CW), axis=2)


def _distances(Q, mem_keys):
    B, D = Q.shape
    M = mem_keys.shape[0]
    nstep = math.ceil(M / TK)
    nt = nstep * (TK // CW)
    S, TM3 = pl.pallas_call(
        functools.partial(_dist_body, M),
        grid=(nstep,),
        in_specs=[
            pl.BlockSpec((B, D), lambda j: (0, 0)),
            pl.BlockSpec((TK, D), lambda j: (j, 0)),
        ],
        out_specs=[
            pl.BlockSpec((B, TK // CW, CW), lambda j: (0, j, 0)),
            pl.BlockSpec((1, B, TK // CW), lambda j: (j, 0, 0)),
        ],
        out_shape=[
            jax.ShapeDtypeStruct((B, nt, CW), jnp.float32),
            jax.ShapeDtypeStruct((nstep, B, TK // CW), jnp.float32),
        ],
    )(Q, mem_keys)
    return S, TM3, nt


def _extract16(vals, ids):
    out_i = []
    big = jnp.int32(2**31 - 1)
    for _ in range(KNN):
        m = jnp.max(vals, axis=1, keepdims=True)
        sel = jnp.where(vals == m, ids, big)
        win = jnp.min(sel, axis=1, keepdims=True)
        out_i.append(win)
        vals = jnp.where(ids == win, -jnp.inf, vals)
    return jnp.concatenate(out_i, axis=1)


def _chunksel_body(nt, rc, tm_ref, gid_ref):
    i = pl.program_id(0)
    vals = tm_ref[...]
    cid = lax.broadcasted_iota(jnp.int32, vals.shape, 1)
    ti = _extract16(vals, cid)
    row = lax.broadcasted_iota(jnp.int32, ti.shape, 0) + i * rc
    gid_ref[...] = row * nt + ti


def _chunk_select(TM, nt):
    B = TM.shape[0]
    rc = min(256, B)
    return pl.pallas_call(
        functools.partial(_chunksel_body, nt, rc),
        grid=(B // rc,),
        in_specs=[pl.BlockSpec((rc, nt), lambda i: (i, 0))],
        out_specs=pl.BlockSpec((rc, KNN), lambda i: (i, 0)),
        out_shape=jax.ShapeDtypeStruct((B, KNN), jnp.int32),
    )(TM)


def _finalsel_body(nt, rc, cv_ref, gid_ref, idx_ref):
    i = pl.program_id(0)
    vals = cv_ref[...]
    gid = gid_ref[...]
    row = lax.broadcasted_iota(jnp.int32, gid.shape, 0) + i * rc
    base = (gid - row * nt) * CW
    lane = lax.broadcasted_iota(jnp.int32, (rc, CW), 1)
    ids = jnp.concatenate(
        [base[:, j:j + 1] + lane for j in range(KNN)], axis=1)
    idx_ref[...] = _extract16(vals, ids)


def _final_select(cand, GID, nt):
    B = GID.shape[0]
    rc = min(256, B)
    return pl.pallas_call(
        functools.partial(_finalsel_body, nt, rc),
        grid=(B // rc,),
        in_specs=[
            pl.BlockSpec((rc, KNN * CW), lambda i: (i, 0)),
            pl.BlockSpec((rc, KNN), lambda i: (i, 0)),
        ],
        out_specs=pl.BlockSpec((rc, KNN), lambda i: (i, 0)),
        out_shape=jax.ShapeDtypeStruct((B, KNN), jnp.int32),
    )(cand, GID)


def _sc_gather_chunks(S2, gid_flat):
    nrow = gid_flat.shape[0]
    per_w = nrow // SC_NW
    ch = 128
    mesh = plsc.VectorSubcoreMesh(core_axis_name="c", subcore_axis_name="s")

    @functools.partial(
        pl.kernel, mesh=mesh,
        out_type=jax.ShapeDtypeStruct((nrow, CW), jnp.float32),
        scratch_types=[
            pltpu.VMEM((ch,), jnp.int32),
            pltpu.VMEM((ch, CW), jnp.float32),
            pltpu.SemaphoreType.DMA,
        ],
    )
    def k(tbl, idxh, outh, idx_v, rows_v, sem):
        wid = lax.axis_index("s") * SC_NC + lax.axis_index("c")
        base = wid * per_w
        for c in range(per_w // ch):
            pltpu.sync_copy(idxh.at[pl.ds(base + c * ch, ch)], idx_v)
            pltpu.async_copy(tbl.at[idx_v], rows_v, sem).wait()
            pltpu.sync_copy(rows_v, outh.at[pl.ds(base + c * ch, ch)])

    return k(S2, gid_flat)


def _sc_gather_rows(mem_keys, mem_vals, idx_flat):
    nrow = idx_flat.shape[0]
    D = mem_keys.shape[1]
    per_w = nrow // SC_NW
    ch = 32
    mesh = plsc.VectorSubcoreMesh(core_axis_name="c", subcore_axis_name="s")

    @functools.partial(
        pl.kernel, mesh=mesh,
        out_type=(jax.ShapeDtypeStruct((nrow, D), jnp.float32),
                  jax.ShapeDtypeStruct((nrow, D), jnp.float32)),
        scratch_types=[
            pltpu.VMEM((per_w,), jnp.int32),
            pltpu.VMEM((2, ch, D), jnp.float32),
            pltpu.VMEM((2, ch, D), jnp.float32),
            pltpu.SemaphoreType.DMA,
            pltpu.SemaphoreType.DMA,
            pltpu.SemaphoreType.DMA,
            pltpu.SemaphoreType.DMA,
        ],
    )
    def k(keys_h, vals_h, idxh, kout, vout, idx_v, krows, vrows,
          ks0, ks1, vs0, vs1):
        wid = lax.axis_index("s") * SC_NC + lax.axis_index("c")
        base = wid * per_w
        ksem = (ks0, ks1)
        vsem = (vs0, vs1)
        pltpu.sync_copy(idxh.at[pl.ds(base, per_w)], idx_v)
        nch = per_w // ch
        cps = [None, None]
        for c in range(nch + 1):
            b = c % 2
            if c < nch:
                isl = idx_v.at[pl.ds(c * ch, ch)]
                cps[b] = (
                    pltpu.async_copy(keys_h.at[isl], krows.at[b], ksem[b]),
                    pltpu.async_copy(vals_h.at[isl], vrows.at[b], vsem[b]),
                )
            if c > 0:
                pb = (c - 1) % 2
                kcp, vcp = cps[pb]
                kcp.wait()
                pltpu.sync_copy(krows.at[pb],
                                kout.at[pl.ds(base + (c - 1) * ch, ch)])
                vcp.wait()
                pltpu.sync_copy(vrows.at[pb],
                                vout.at[pl.ds(base + (c - 1) * ch, ch)])

    return k(mem_keys, mem_vals, idx_flat)


def _tail_body(rt, q_ref, kn_ref, vn_ref, wqc_ref, wkc_ref, wvc_ref,
               bc_ref, wo_ref, bo_ref, w1_ref, b1_ref, w2_ref,
               b2_ref, w3_ref, b3_ref, out_ref):
    D = q_ref.shape[1]
    dh = D // H
    Qb = q_ref[...]

    q2 = _dot_t(Qb, wqc_ref[...], TAIL_PREC) + bc_ref[0:1, :]
    k2 = (_dot_t(kn_ref[...].reshape(KNN * rt, D), wkc_ref[...], TAIL_PREC)
          + bc_ref[1:2, :]).reshape(KNN, rt, D)
    v2 = (_dot_t(vn_ref[...].reshape(KNN * rt, D), wvc_ref[...], TAIL_PREC)
          + bc_ref[2:3, :]).reshape(KNN, rt, D)

    prod = k2 * q2[None, :, :]
    logits = jnp.concatenate(
        [jnp.sum(prod[:, :, h * dh:(h + 1) * dh], axis=2, keepdims=True)
         for h in range(H)], axis=2)
    logits = logits * (1.0 / math.sqrt(dh))
    mx = jnp.max(logits, axis=0, keepdims=True)
    p = jnp.exp(logits - mx)
    p = p / jnp.sum(p, axis=0, keepdims=True)
    pb = jnp.concatenate(
        [jnp.broadcast_to(p[:, :, h:h + 1], (KNN, rt, dh)) for h in range(H)],
        axis=2)
    ctx = jnp.sum(pb * v2, axis=0)

    prior = _dot_t(ctx, wo_ref[...], TAIL_PREC) + bo_ref[...]
    feat1 = _dot_t(Qb, w1_ref[...], TAIL_PREC) + b1_ref[...]
    feat2 = _dot_t(prior, w2_ref[...], TAIL_PREC) + b2_ref[...]
    feat = jnp.concatenate([feat1, feat2], axis=1)
    feat = feat * 0.5 * (1.0 + lax.erf(feat * (1.0 / math.sqrt(2.0))))
    out_ref[...] = _dot_t(feat, w3_ref[...], TAIL_PREC) + b3_ref[...]


def _tail(Q, KnT, VnT, Wqc, Wkc, Wvc, bc, Wo, bo, W1, b1, W2, b2, W3, b3):
    B, D = Q.shape
    rt = min(128, B)
    o3 = W3.shape[0]
    full = lambda a: pl.BlockSpec(a.shape, lambda i: (0,) * a.ndim)
    return pl.pallas_call(
        functools.partial(_tail_body, rt),
        grid=(B // rt,),
        in_specs=[
            pl.BlockSpec((rt, D), lambda i: (i, 0)),
            pl.BlockSpec((KNN, rt, D), lambda i: (0, i, 0)),
            pl.BlockSpec((KNN, rt, D), lambda i: (0, i, 0)),
            full(Wqc), full(Wkc), full(Wvc), full(bc), full(Wo), full(bo),
            full(W1), full(b1), full(W2), full(b2), full(W3), full(b3),
        ],
        out_specs=pl.BlockSpec((rt, o3), lambda i: (i, 0)),
        out_shape=jax.ShapeDtypeStruct((B, o3), jnp.float32),
    )(Q, KnT, VnT, Wqc, Wkc, Wvc, bc, Wo, bo, W1, b1, W2, b2, W3, b3)


def kernel(x, mem_keys, mem_vals, W_img, b_img, Wq, bq, Wk, bk, Wv, bv,
           W_in, b_in, Wo, bo, W1, b1, W2, b2, W3, b3):
    B, D = x.shape
    M = mem_keys.shape[0]

    b_pre = jnp.stack([bq, bk, bv], axis=0)
    b_in3 = b_in.reshape(3, D)
    Q, Wqc, Wkc, Wvc, bc = _project(x, W_img, b_img, Wq, Wk, Wv, W_in,
                                    b_pre, b_in3)
    S, TM3, nt = _distances(Q, mem_keys)
    TM = TM3.transpose(1, 0, 2).reshape(B, nt)
    GID = _chunk_select(TM, nt)
    cand = _sc_gather_chunks(S.reshape(B * nt, CW), GID.reshape(-1))
    idx = _final_select(cand.reshape(B, KNN * CW), GID, nt)

    idx_t = idx.T.reshape(-1)
    KnF, VnF = _sc_gather_rows(mem_keys, mem_vals, idx_t)
    KnT = KnF.reshape(KNN, B, D)
    VnT = VnF.reshape(KNN, B, D)

    feat = _tail(Q, KnT, VnT, Wqc, Wkc, Wvc, bc, Wo,
                 bo.reshape(1, D), W1, b1.reshape(1, -1), W2,
                 b2.reshape(1, -1), W3, b3.reshape(1, -1))
    return feat, Q


# speedup vs baseline: 8.5664x; 1.0996x over previous
import functools
import math

import jax
import jax.numpy as jnp
from jax import lax
from jax.experimental import pallas as pl
from jax.experimental.pallas import tpu as pltpu
from jax.experimental.pallas import tpu_sc as plsc

KNN = 16
H = 8
CW = 128
TK = 2048
NEG = -1e30

SC_NC = 2
SC_NS = 16
SC_NW = SC_NC * SC_NS

DIST_PREC = lax.Precision.DEFAULT
TAIL_PREC = lax.Precision.DEFAULT


def _dot_t(a, b, prec):
    return lax.dot_general(a, b, (((1,), (1,)), ((), ())), precision=prec,
                           preferred_element_type=jnp.float32)


def _dot(a, b, prec):
    return lax.dot_general(a, b, (((1,), (0,)), ((), ())), precision=prec,
                           preferred_element_type=jnp.float32)


def _proj_body(x_ref, w_ref, b_ref, wq_ref, wk_ref, wv_ref, win_ref,
               bpre_ref, bin_ref, q_ref, wqc_ref, wkc_ref, wvc_ref, bc_ref):
    D = x_ref.shape[1]
    q_ref[...] = _dot_t(x_ref[...], w_ref[...], DIST_PREC) + b_ref[...]
    wiq = win_ref[0:D, :]
    wik = win_ref[D:2 * D, :]
    wiv = win_ref[2 * D:3 * D, :]
    wqc_ref[...] = _dot(wiq, wq_ref[...], TAIL_PREC)
    wkc_ref[...] = _dot(wik, wk_ref[...], TAIL_PREC)
    wvc_ref[...] = _dot(wiv, wv_ref[...], TAIL_PREC)
    bc_ref[...] = jnp.concatenate(
        [_dot_t(bpre_ref[0:1, :], wiq, TAIL_PREC) + bin_ref[0:1, :],
         _dot_t(bpre_ref[1:2, :], wik, TAIL_PREC) + bin_ref[1:2, :],
         _dot_t(bpre_ref[2:3, :], wiv, TAIL_PREC) + bin_ref[2:3, :]], axis=0)


def _project(x, W_img, b_img, Wq, Wk, Wv, W_in, b_pre, b_in3):
    B, D = x.shape
    f32 = jnp.float32
    return pl.pallas_call(
        _proj_body,
        out_shape=[jax.ShapeDtypeStruct((B, D), f32),
                   jax.ShapeDtypeStruct((D, D), f32),
                   jax.ShapeDtypeStruct((D, D), f32),
                   jax.ShapeDtypeStruct((D, D), f32),
                   jax.ShapeDtypeStruct((3, D), f32)],
    )(x, W_img, b_img.reshape(1, D), Wq, Wk, Wv, W_in, b_pre, b_in3)


def _dist_body(M, q_ref, k_ref, s_ref, tm_ref):
    j = pl.program_id(0)
    Qb = q_ref[...]
    Kb = k_ref[...]
    B = Qb.shape[0]
    qsq = jnp.sum(Qb * Qb, axis=1, keepdims=True)
    ksq = jnp.sum(Kb * Kb, axis=1)
    qk = _dot_t(Qb, Kb, DIST_PREC)
    dist2 = qsq - 2.0 * qk + ksq[None, :]
    col = lax.broadcasted_iota(jnp.int32, (B, TK), 1) + j * TK
    s = jnp.where(col < M, -dist2, NEG)
    s_ref[...] = s.reshape(B, TK // CW, CW)
    tm_ref[0] = jnp.max(s.reshape(B, TK // CW, CW), axis=2)


def _distances(Q, mem_keys):
    B, D = Q.shape
    M = mem_keys.shape[0]
    nstep = math.ceil(M / TK)
    nt = nstep * (TK // CW)
    S, TM3 = pl.pallas_call(
        functools.partial(_dist_body, M),
        grid=(nstep,),
        in_specs=[
            pl.BlockSpec((B, D), lambda j: (0, 0)),
            pl.BlockSpec((TK, D), lambda j: (j, 0)),
        ],
        out_specs=[
            pl.BlockSpec((B, TK // CW, CW), lambda j: (0, j, 0)),
            pl.BlockSpec((1, B, TK // CW), lambda j: (j, 0, 0)),
        ],
        out_shape=[
            jax.ShapeDtypeStruct((B, nt, CW), jnp.float32),
            jax.ShapeDtypeStruct((nstep, B, TK // CW), jnp.float32),
        ],
    )(Q, mem_keys)
    return S, TM3, nt


def _extract16(vals, ids):
    out_i = []
    big = jnp.int32(2**31 - 1)
    for _ in range(KNN):
        m = jnp.max(vals, axis=1, keepdims=True)
        sel = jnp.where(vals == m, ids, big)
        win = jnp.min(sel, axis=1, keepdims=True)
        out_i.append(win)
        vals = jnp.where(ids == win, -jnp.inf, vals)
    return jnp.concatenate(out_i, axis=1)


def _chunksel_body(nt, rc, tm_ref, gid_ref):
    i = pl.program_id(0)
    vals = tm_ref[...]
    cid = lax.broadcasted_iota(jnp.int32, vals.shape, 1)
    ti = _extract16(vals, cid)
    row = lax.broadcasted_iota(jnp.int32, ti.shape, 0) + i * rc
    gid_ref[...] = row * nt + ti


def _chunk_select(TM, nt):
    B = TM.shape[0]
    rc = min(256, B)
    return pl.pallas_call(
        functools.partial(_chunksel_body, nt, rc),
        grid=(B // rc,),
        in_specs=[pl.BlockSpec((rc, nt), lambda i: (i, 0))],
        out_specs=pl.BlockSpec((rc, KNN), lambda i: (i, 0)),
        out_shape=jax.ShapeDtypeStruct((B, KNN), jnp.int32),
    )(TM)


def _finalsel_body(nt, rc, cv_ref, gid_ref, idx_ref):
    i = pl.program_id(0)
    vals = cv_ref[...]
    gid = gid_ref[...]
    row = lax.broadcasted_iota(jnp.int32, gid.shape, 0) + i * rc
    base = (gid - row * nt) * CW
    lane = lax.broadcasted_iota(jnp.int32, (rc, CW), 1)
    ids = jnp.concatenate(
        [base[:, j:j + 1] + lane for j in range(KNN)], axis=1)
    idx_ref[...] = _extract16(vals, ids)


def _final_select(cand, GID, nt):
    B = GID.shape[0]
    rc = min(256, B)
    return pl.pallas_call(
        functools.partial(_finalsel_body, nt, rc),
        grid=(B // rc,),
        in_specs=[
            pl.BlockSpec((rc, KNN * CW), lambda i: (i, 0)),
            pl.BlockSpec((rc, KNN), lambda i: (i, 0)),
        ],
        out_specs=pl.BlockSpec((rc, KNN), lambda i: (i, 0)),
        out_shape=jax.ShapeDtypeStruct((B, KNN), jnp.int32),
    )(cand, GID)


def _sc_gather_chunks(S2, gid_flat):
    nrow = gid_flat.shape[0]
    per_w = nrow // SC_NW
    ch = 128
    mesh = plsc.VectorSubcoreMesh(core_axis_name="c", subcore_axis_name="s")

    @functools.partial(
        pl.kernel, mesh=mesh,
        out_type=jax.ShapeDtypeStruct((nrow, CW), jnp.float32),
        scratch_types=[
            pltpu.VMEM((ch,), jnp.int32),
            pltpu.VMEM((ch, CW), jnp.float32),
            pltpu.SemaphoreType.DMA,
        ],
    )
    def k(tbl, idxh, outh, idx_v, rows_v, sem):
        wid = lax.axis_index("s") * SC_NC + lax.axis_index("c")
        base = wid * per_w
        for c in range(per_w // ch):
            pltpu.sync_copy(idxh.at[pl.ds(base + c * ch, ch)], idx_v)
            pltpu.async_copy(tbl.at[idx_v], rows_v, sem).wait()
            pltpu.sync_copy(rows_v, outh.at[pl.ds(base + c * ch, ch)])

    return k(S2, gid_flat)


def _sc_gather_rows(mem_keys, mem_vals, idx_flat):
    nrow = idx_flat.shape[0]
    D = mem_keys.shape[1]
    per_w = nrow // SC_NW
    ch = 64
    mesh = plsc.VectorSubcoreMesh(core_axis_name="c", subcore_axis_name="s")

    @functools.partial(
        pl.kernel, mesh=mesh,
        out_type=(jax.ShapeDtypeStruct((nrow, D), jnp.float32),
                  jax.ShapeDtypeStruct((nrow, D), jnp.float32)),
        scratch_types=[
            pltpu.VMEM((per_w,), jnp.int32),
            pltpu.VMEM((ch, D), jnp.float32),
            pltpu.VMEM((ch, D), jnp.float32),
            pltpu.SemaphoreType.DMA,
            pltpu.SemaphoreType.DMA,
        ],
    )
    def k(keys_h, vals_h, idxh, kout, vout, idx_v, krows, vrows, ksem, vsem):
        wid = lax.axis_index("s") * SC_NC + lax.axis_index("c")
        base = wid * per_w
        pltpu.sync_copy(idxh.at[pl.ds(base, per_w)], idx_v)
        for c in range(per_w // ch):
            isl = idx_v.at[pl.ds(c * ch, ch)]
            kcp = pltpu.async_copy(keys_h.at[isl], krows, ksem)
            vcp = pltpu.async_copy(vals_h.at[isl], vrows, vsem)
            kcp.wait()
            pltpu.sync_copy(krows, kout.at[pl.ds(base + c * ch, ch)])
            vcp.wait()
            pltpu.sync_copy(vrows, vout.at[pl.ds(base + c * ch, ch)])

    return k(mem_keys, mem_vals, idx_flat)


def _tail_body(rt, q_ref, kn_ref, vn_ref, wqc_ref, wkc_ref, wvc_ref,
               bc_ref, wo_ref, bo_ref, w1_ref, b1_ref, w2_ref,
               b2_ref, w3_ref, b3_ref, out_ref):
    D = q_ref.shape[1]
    dh = D // H
    Qb = q_ref[...]

    q2 = _dot_t(Qb, wqc_ref[...], TAIL_PREC) + bc_ref[0:1, :]
    Kn = kn_ref[...]
    Vn = vn_ref[...]

    lgs = []
    for h in range(H):
        sl = slice(h * dh, (h + 1) * dh)
        gh = _dot(q2[:, sl], wkc_ref[sl, :], TAIL_PREC)
        bh = jnp.sum(q2[:, sl] * bc_ref[1:2, sl], axis=1)
        lh = jnp.sum(Kn * gh[None, :, :], axis=2, keepdims=True)
        lgs.append(lh + bh[None, :, None])
    logits = jnp.concatenate(lgs, axis=2)
    logits = logits * (1.0 / math.sqrt(dh))
    mx = jnp.max(logits, axis=0, keepdims=True)
    p = jnp.exp(logits - mx)
    p = p / jnp.sum(p, axis=0, keepdims=True)
    pb = jnp.concatenate(
        [jnp.broadcast_to(p[:, :, h:h + 1], (KNN, rt, dh)) for h in range(H)],
        axis=2)
    vsum = jnp.sum(pb * Vn, axis=0)
    ctx = _dot_t(vsum, wvc_ref[...], TAIL_PREC) + bc_ref[2:3, :]

    prior = _dot_t(ctx, wo_ref[...], TAIL_PREC) + bo_ref[...]
    feat1 = _dot_t(Qb, w1_ref[...], TAIL_PREC) + b1_ref[...]
    feat2 = _dot_t(prior, w2_ref[...], TAIL_PREC) + b2_ref[...]
    feat = jnp.concatenate([feat1, feat2], axis=1)
    feat = feat * 0.5 * (1.0 + lax.erf(feat * (1.0 / math.sqrt(2.0))))
    out_ref[...] = _dot_t(feat, w3_ref[...], TAIL_PREC) + b3_ref[...]


def _tail(Q, KnT, VnT, Wqc, Wkc, Wvc, bc, Wo, bo, W1, b1, W2, b2, W3, b3):
    B, D = Q.shape
    rt = min(128, B)
    o3 = W3.shape[0]
    full = lambda a: pl.BlockSpec(a.shape, lambda i: (0,) * a.ndim)
    return pl.pallas_call(
        functools.partial(_tail_body, rt),
        grid=(B // rt,),
        in_specs=[
            pl.BlockSpec((rt, D), lambda i: (i, 0)),
            pl.BlockSpec((KNN, rt, D), lambda i: (0, i, 0)),
            pl.BlockSpec((KNN, rt, D), lambda i: (0, i, 0)),
            full(Wqc), full(Wkc), full(Wvc), full(bc), full(Wo), full(bo),
            full(W1), full(b1), full(W2), full(b2), full(W3), full(b3),
        ],
        out_specs=pl.BlockSpec((rt, o3), lambda i: (i, 0)),
        out_shape=jax.ShapeDtypeStruct((B, o3), jnp.float32),
    )(Q, KnT, VnT, Wqc, Wkc, Wvc, bc, Wo, bo, W1, b1, W2, b2, W3, b3)


def kernel(x, mem_keys, mem_vals, W_img, b_img, Wq, bq, Wk, bk, Wv, bv,
           W_in, b_in, Wo, bo, W1, b1, W2, b2, W3, b3):
    B, D = x.shape
    M = mem_keys.shape[0]

    b_pre = jnp.stack([bq, bk, bv], axis=0)
    b_in3 = b_in.reshape(3, D)
    Q, Wqc, Wkc, Wvc, bc = _project(x, W_img, b_img, Wq, Wk, Wv, W_in,
                                    b_pre, b_in3)
    S, TM3, nt = _distances(Q, mem_keys)
    TM = TM3.transpose(1, 0, 2).reshape(B, nt)
    GID = _chunk_select(TM, nt)
    cand = _sc_gather_chunks(S.reshape(B * nt, CW), GID.reshape(-1))
    idx = _final_select(cand.reshape(B, KNN * CW), GID, nt)

    idx_t = idx.T.reshape(-1)
    KnF, VnF = _sc_gather_rows(mem_keys, mem_vals, idx_t)
    KnT = KnF.reshape(KNN, B, D)
    VnT = VnF.reshape(KNN, B, D)

    feat = _tail(Q, KnT, VnT, Wqc, Wkc, Wvc, bc, Wo,
                 bo.reshape(1, D), W1, b1.reshape(1, -1), W2,
                 b2.reshape(1, -1), W3, b3.reshape(1, -1))
    return feat, Q


# speedup vs baseline: 8.7893x; 1.0260x over previous
import functools
import math

import jax
import jax.numpy as jnp
from jax import lax
from jax.experimental import pallas as pl
from jax.experimental.pallas import tpu as pltpu
from jax.experimental.pallas import tpu_sc as plsc

KNN = 16
H = 8
CW = 128
TK = 3072
NEG = -1e30

SC_NC = 2
SC_NS = 16
SC_NW = SC_NC * SC_NS

DIST_PREC = lax.Precision.DEFAULT
TAIL_PREC = lax.Precision.DEFAULT


def _dot_t(a, b, prec):
    return lax.dot_general(a, b, (((1,), (1,)), ((), ())), precision=prec,
                           preferred_element_type=jnp.float32)


def _dot(a, b, prec):
    return lax.dot_general(a, b, (((1,), (0,)), ((), ())), precision=prec,
                           preferred_element_type=jnp.float32)


def _proj_body(x_ref, w_ref, b_ref, wq_ref, wk_ref, wv_ref, win_ref,
               bpre_ref, bin_ref, q_ref, wqc_ref, wkc_ref, wvc_ref, bc_ref):
    D = x_ref.shape[1]
    q_ref[...] = _dot_t(x_ref[...], w_ref[...], DIST_PREC) + b_ref[...]
    wiq = win_ref[0:D, :]
    wik = win_ref[D:2 * D, :]
    wiv = win_ref[2 * D:3 * D, :]
    wqc_ref[...] = _dot(wiq, wq_ref[...], TAIL_PREC)
    wkc_ref[...] = _dot(wik, wk_ref[...], TAIL_PREC)
    wvc_ref[...] = _dot(wiv, wv_ref[...], TAIL_PREC)
    bc_ref[...] = jnp.concatenate(
        [_dot_t(bpre_ref[0:1, :], wiq, TAIL_PREC) + bin_ref[0:1, :],
         _dot_t(bpre_ref[1:2, :], wik, TAIL_PREC) + bin_ref[1:2, :],
         _dot_t(bpre_ref[2:3, :], wiv, TAIL_PREC) + bin_ref[2:3, :]], axis=0)


def _project(x, W_img, b_img, Wq, Wk, Wv, W_in, b_pre, b_in3):
    B, D = x.shape
    f32 = jnp.float32
    return pl.pallas_call(
        _proj_body,
        out_shape=[jax.ShapeDtypeStruct((B, D), f32),
                   jax.ShapeDtypeStruct((D, D), f32),
                   jax.ShapeDtypeStruct((D, D), f32),
                   jax.ShapeDtypeStruct((D, D), f32),
                   jax.ShapeDtypeStruct((3, D), f32)],
    )(x, W_img, b_img.reshape(1, D), Wq, Wk, Wv, W_in, b_pre, b_in3)


def _dist_body(M, q_ref, k_ref, s_ref, tm_ref):
    j = pl.program_id(0)
    Qb = q_ref[...]
    Kb = k_ref[...]
    B = Qb.shape[0]
    nc = TK // CW
    kvalid = (lax.broadcasted_iota(jnp.int32, (TK, 1), 0) + j * TK) < M
    Kb = jnp.where(kvalid, Kb, 0.0)
    qsq = jnp.sum(Qb * Qb, axis=1, keepdims=True)
    ksq = jnp.sum(Kb * Kb, axis=1, keepdims=True)
    ksq = jnp.where(kvalid, ksq, -NEG)
    qk = _dot_t(Qb, Kb, DIST_PREC)
    s = -(qsq - 2.0 * qk + ksq[:, 0][None, :])
    s_ref[...] = s.reshape(B, nc, CW)
    tm_ref[0] = jnp.max(s.reshape(B, nc, CW), axis=2)


def _distances(Q, mem_keys):
    B, D = Q.shape
    M = mem_keys.shape[0]
    nstep = math.ceil(M / TK)
    nt = nstep * (TK // CW)
    S, TM3 = pl.pallas_call(
        functools.partial(_dist_body, M),
        grid=(nstep,),
        in_specs=[
            pl.BlockSpec((B, D), lambda j: (0, 0)),
            pl.BlockSpec((TK, D), lambda j: (j, 0)),
        ],
        out_specs=[
            pl.BlockSpec((B, TK // CW, CW), lambda j: (0, j, 0)),
            pl.BlockSpec((1, B, TK // CW), lambda j: (j, 0, 0)),
        ],
        out_shape=[
            jax.ShapeDtypeStruct((B, nt, CW), jnp.float32),
            jax.ShapeDtypeStruct((nstep, B, TK // CW), jnp.float32),
        ],
    )(Q, mem_keys)
    return S, TM3, nt


def _chunksel_body(nt, rc, tm_ref, gid_ref):
    i = pl.program_id(0)
    vals = tm_ref[...]
    cid = lax.broadcasted_iota(jnp.int32, vals.shape, 1)
    ti = _extract16(vals, cid)
    row = lax.broadcasted_iota(jnp.int32, ti.shape, 0) + i * rc
    gid_ref[...] = row * nt + ti


def _chunk_select(TM, nt):
    B = TM.shape[0]
    rc = min(256, B)
    return pl.pallas_call(
        functools.partial(_chunksel_body, nt, rc),
        grid=(B // rc,),
        in_specs=[pl.BlockSpec((rc, nt), lambda i: (i, 0))],
        out_specs=pl.BlockSpec((rc, KNN), lambda i: (i, 0)),
        out_shape=jax.ShapeDtypeStruct((B, KNN), jnp.int32),
    )(TM)


def _extract16(vals, ids):
    out_i = []
    big = jnp.int32(2**31 - 1)
    for _ in range(KNN):
        m = jnp.max(vals, axis=1, keepdims=True)
        sel = jnp.where(vals == m, ids, big)
        win = jnp.min(sel, axis=1, keepdims=True)
        out_i.append(win)
        vals = jnp.where(ids == win, -jnp.inf, vals)
    return jnp.concatenate(out_i, axis=1)


def _finalsel_body(nt, rc, cv_ref, gid_ref, idx_ref):
    i = pl.program_id(0)
    vals = cv_ref[...]
    gid = gid_ref[...]
    row = lax.broadcasted_iota(jnp.int32, gid.shape, 0) + i * rc
    base = (gid - row * nt) * CW
    lane = lax.broadcasted_iota(jnp.int32, (rc, CW), 1)
    ids = jnp.concatenate(
        [base[:, j:j + 1] + lane for j in range(KNN)], axis=1)
    idx_ref[...] = _extract16(vals, ids)


def _final_select(cand, GID, nt):
    B = GID.shape[0]
    rc = min(256, B)
    return pl.pallas_call(
        functools.partial(_finalsel_body, nt, rc),
        grid=(B // rc,),
        in_specs=[
            pl.BlockSpec((rc, KNN * CW), lambda i: (i, 0)),
            pl.BlockSpec((rc, KNN), lambda i: (i, 0)),
        ],
        out_specs=pl.BlockSpec((rc, KNN), lambda i: (i, 0)),
        out_shape=jax.ShapeDtypeStruct((B, KNN), jnp.int32),
    )(cand, GID)


def _sc_gather_chunks(S2, gid_flat):
    nrow = gid_flat.shape[0]
    per_w = nrow // SC_NW
    ch = 128
    mesh = plsc.VectorSubcoreMesh(core_axis_name="c", subcore_axis_name="s")

    @functools.partial(
        pl.kernel, mesh=mesh,
        out_type=jax.ShapeDtypeStruct((nrow, CW), jnp.float32),
        scratch_types=[
            pltpu.VMEM((ch,), jnp.int32),
            pltpu.VMEM((ch, CW), jnp.float32),
            pltpu.SemaphoreType.DMA,
        ],
    )
    def k(tbl, idxh, outh, idx_v, rows_v, sem):
        wid = lax.axis_index("s") * SC_NC + lax.axis_index("c")
        base = wid * per_w
        for c in range(per_w // ch):
            pltpu.sync_copy(idxh.at[pl.ds(base + c * ch, ch)], idx_v)
            pltpu.async_copy(tbl.at[idx_v], rows_v, sem).wait()
            pltpu.sync_copy(rows_v, outh.at[pl.ds(base + c * ch, ch)])

    return k(S2, gid_flat)


def _sc_gather_rows(mem_keys, mem_vals, idx_flat):
    nrow = idx_flat.shape[0]
    D = mem_keys.shape[1]
    per_w = nrow // SC_NW
    ch = 64
    mesh = plsc.VectorSubcoreMesh(core_axis_name="c", subcore_axis_name="s")

    @functools.partial(
        pl.kernel, mesh=mesh,
        out_type=(jax.ShapeDtypeStruct((nrow, D), jnp.float32),
                  jax.ShapeDtypeStruct((nrow, D), jnp.float32)),
        scratch_types=[
            pltpu.VMEM((per_w,), jnp.int32),
            pltpu.VMEM((ch, D), jnp.float32),
            pltpu.VMEM((ch, D), jnp.float32),
            pltpu.SemaphoreType.DMA,
            pltpu.SemaphoreType.DMA,
        ],
    )
    def k(keys_h, vals_h, idxh, kout, vout, idx_v, krows, vrows, ksem, vsem):
        wid = lax.axis_index("s") * SC_NC + lax.axis_index("c")
        base = wid * per_w
        pltpu.sync_copy(idxh.at[pl.ds(base, per_w)], idx_v)
        for c in range(per_w // ch):
            isl = idx_v.at[pl.ds(c * ch, ch)]
            kcp = pltpu.async_copy(keys_h.at[isl], krows, ksem)
            vcp = pltpu.async_copy(vals_h.at[isl], vrows, vsem)
            kcp.wait()
            pltpu.sync_copy(krows, kout.at[pl.ds(base + c * ch, ch)])
            vcp.wait()
            pltpu.sync_copy(vrows, vout.at[pl.ds(base + c * ch, ch)])

    return k(mem_keys, mem_vals, idx_flat)


def _tail_body(rt, q_ref, kn_ref, vn_ref, wqc_ref, wkc_ref, wvc_ref,
               bc_ref, wo_ref, bo_ref, w1_ref, b1_ref, w2_ref,
               b2_ref, w3_ref, b3_ref, out_ref):
    D = q_ref.shape[1]
    dh = D // H
    Qb = q_ref[...]

    q2 = _dot_t(Qb, wqc_ref[...], TAIL_PREC) + bc_ref[0:1, :]
    Kn = kn_ref[...]
    Vn = vn_ref[...]

    lgs = []
    for h in range(H):
        sl = slice(h * dh, (h + 1) * dh)
        gh = _dot(q2[:, sl], wkc_ref[sl, :], TAIL_PREC)
        bh = jnp.sum(q2[:, sl] * bc_ref[1:2, sl], axis=1)
        lh = jnp.sum(Kn * gh[None, :, :], axis=2, keepdims=True)
        lgs.append(lh + bh[None, :, None])
    logits = jnp.concatenate(lgs, axis=2)
    logits = logits * (1.0 / math.sqrt(dh))
    mx = jnp.max(logits, axis=0, keepdims=True)
    p = jnp.exp(logits - mx)
    p = p / jnp.sum(p, axis=0, keepdims=True)
    pb = jnp.concatenate(
        [jnp.broadcast_to(p[:, :, h:h + 1], (KNN, rt, dh)) for h in range(H)],
        axis=2)
    vsum = jnp.sum(pb * Vn, axis=0)
    ctx = _dot_t(vsum, wvc_ref[...], TAIL_PREC) + bc_ref[2:3, :]

    prior = _dot_t(ctx, wo_ref[...], TAIL_PREC) + bo_ref[...]
    feat1 = _dot_t(Qb, w1_ref[...], TAIL_PREC) + b1_ref[...]
    feat2 = _dot_t(prior, w2_ref[...], TAIL_PREC) + b2_ref[...]
    feat = jnp.concatenate([feat1, feat2], axis=1)
    feat = feat * 0.5 * (1.0 + lax.erf(feat * (1.0 / math.sqrt(2.0))))
    out_ref[...] = _dot_t(feat, w3_ref[...], TAIL_PREC) + b3_ref[...]


def _tail(Q, KnT, VnT, Wqc, Wkc, Wvc, bc, Wo, bo, W1, b1, W2, b2, W3, b3):
    B, D = Q.shape
    rt = min(128, B)
    o3 = W3.shape[0]
    full = lambda a: pl.BlockSpec(a.shape, lambda i: (0,) * a.ndim)
    return pl.pallas_call(
        functools.partial(_tail_body, rt),
        grid=(B // rt,),
        in_specs=[
            pl.BlockSpec((rt, D), lambda i: (i, 0)),
            pl.BlockSpec((KNN, rt, D), lambda i: (0, i, 0)),
            pl.BlockSpec((KNN, rt, D), lambda i: (0, i, 0)),
            full(Wqc), full(Wkc), full(Wvc), full(bc), full(Wo), full(bo),
            full(W1), full(b1), full(W2), full(b2), full(W3), full(b3),
        ],
        out_specs=pl.BlockSpec((rt, o3), lambda i: (i, 0)),
        out_shape=jax.ShapeDtypeStruct((B, o3), jnp.float32),
    )(Q, KnT, VnT, Wqc, Wkc, Wvc, bc, Wo, bo, W1, b1, W2, b2, W3, b3)


def kernel(x, mem_keys, mem_vals, W_img, b_img, Wq, bq, Wk, bk, Wv, bv,
           W_in, b_in, Wo, bo, W1, b1, W2, b2, W3, b3):
    B, D = x.shape
    M = mem_keys.shape[0]

    b_pre = jnp.stack([bq, bk, bv], axis=0)
    b_in3 = b_in.reshape(3, D)
    Q, Wqc, Wkc, Wvc, bc = _project(x, W_img, b_img, Wq, Wk, Wv, W_in,
                                    b_pre, b_in3)
    S, TM3, nt = _distances(Q, mem_keys)
    TM = TM3.transpose(1, 0, 2).reshape(B, nt)
    GID = _chunk_select(TM, nt)
    cand = _sc_gather_chunks(S.reshape(B * nt, CW), GID.reshape(-1))
    idx = _final_select(cand.reshape(B, KNN * CW), GID, nt)

    idx_t = idx.T.reshape(-1)
    KnF, VnF = _sc_gather_rows(mem_keys, mem_vals, idx_t)
    KnT = KnF.reshape(KNN, B, D)
    VnT = VnF.reshape(KNN, B, D)

    feat = _tail(Q, KnT, VnT, Wqc, Wkc, Wvc, bc, Wo,
                 bo.reshape(1, D), W1, b1.reshape(1, -1), W2,
                 b2.reshape(1, -1), W3, b3.reshape(1, -1))
    return feat, Q


# speedup vs baseline: 8.8076x; 1.0021x over previous
import functools
import math

import jax
import jax.numpy as jnp
from jax import lax
from jax.experimental import pallas as pl
from jax.experimental.pallas import tpu as pltpu
from jax.experimental.pallas import tpu_sc as plsc

KNN = 16
H = 8
CW = 128
TK = 4096
_BIG_VMEM = pltpu.CompilerParams(vmem_limit_bytes=100 * 1024 * 1024)
NEG = -1e30

SC_NC = 2
SC_NS = 16
SC_NW = SC_NC * SC_NS

DIST_PREC = lax.Precision.DEFAULT
TAIL_PREC = lax.Precision.DEFAULT


def _dot_t(a, b, prec):
    return lax.dot_general(a, b, (((1,), (1,)), ((), ())), precision=prec,
                           preferred_element_type=jnp.float32)


def _dot(a, b, prec):
    return lax.dot_general(a, b, (((1,), (0,)), ((), ())), precision=prec,
                           preferred_element_type=jnp.float32)


def _proj_body(x_ref, w_ref, b_ref, wq_ref, wk_ref, wv_ref, win_ref,
               bpre_ref, bin_ref, q_ref, wqc_ref, wkc_ref, wvc_ref, bc_ref):
    D = x_ref.shape[1]
    q_ref[...] = _dot_t(x_ref[...], w_ref[...], DIST_PREC) + b_ref[...]
    wiq = win_ref[0:D, :]
    wik = win_ref[D:2 * D, :]
    wiv = win_ref[2 * D:3 * D, :]
    wqc_ref[...] = _dot(wiq, wq_ref[...], TAIL_PREC)
    wkc_ref[...] = _dot(wik, wk_ref[...], TAIL_PREC)
    wvc_ref[...] = _dot(wiv, wv_ref[...], TAIL_PREC)
    bc_ref[...] = jnp.concatenate(
        [_dot_t(bpre_ref[0:1, :], wiq, TAIL_PREC) + bin_ref[0:1, :],
         _dot_t(bpre_ref[1:2, :], wik, TAIL_PREC) + bin_ref[1:2, :],
         _dot_t(bpre_ref[2:3, :], wiv, TAIL_PREC) + bin_ref[2:3, :]], axis=0)


def _project(x, W_img, b_img, Wq, Wk, Wv, W_in, b_pre, b_in3):
    B, D = x.shape
    f32 = jnp.float32
    return pl.pallas_call(
        _proj_body,
        out_shape=[jax.ShapeDtypeStruct((B, D), f32),
                   jax.ShapeDtypeStruct((D, D), f32),
                   jax.ShapeDtypeStruct((D, D), f32),
                   jax.ShapeDtypeStruct((D, D), f32),
                   jax.ShapeDtypeStruct((3, D), f32)],
    )(x, W_img, b_img.reshape(1, D), Wq, Wk, Wv, W_in, b_pre, b_in3)


def _dist_body(M, q_ref, k_ref, s_ref, tm_ref):
    j = pl.program_id(0)
    Qb = q_ref[...]
    Kb = k_ref[...]
    B = Qb.shape[0]
    nc = TK // CW
    kvalid = (lax.broadcasted_iota(jnp.int32, (TK, 1), 0) + j * TK) < M
    Kb = jnp.where(kvalid, Kb, 0.0)
    qsq = jnp.sum(Qb * Qb, axis=1, keepdims=True)
    ksq = jnp.sum(Kb * Kb, axis=1, keepdims=True)
    ksq = jnp.where(kvalid, ksq, -NEG)
    qk = _dot_t(Qb, Kb, DIST_PREC)
    s = -(qsq - 2.0 * qk + ksq[:, 0][None, :])
    s_ref[...] = s.reshape(B, nc, CW)
    tm_ref[0] = jnp.max(s.reshape(B, nc, CW), axis=2)


def _distances(Q, mem_keys):
    B, D = Q.shape
    M = mem_keys.shape[0]
    nstep = math.ceil(M / TK)
    nt = nstep * (TK // CW)
    S, TM3 = pl.pallas_call(
        functools.partial(_dist_body, M),
        grid=(nstep,),
        in_specs=[
            pl.BlockSpec((B, D), lambda j: (0, 0)),
            pl.BlockSpec((TK, D), lambda j: (j, 0)),
        ],
        out_specs=[
            pl.BlockSpec((B, TK // CW, CW), lambda j: (0, j, 0)),
            pl.BlockSpec((1, B, TK // CW), lambda j: (j, 0, 0)),
        ],
        out_shape=[
            jax.ShapeDtypeStruct((B, nt, CW), jnp.float32),
            jax.ShapeDtypeStruct((nstep, B, TK // CW), jnp.float32),
        ],
        compiler_params=_BIG_VMEM,
    )(Q, mem_keys)
    return S, TM3, nt


def _chunksel_body(nt, rc, tm_ref, gid_ref):
    i = pl.program_id(0)
    vals = tm_ref[...]
    cid = lax.broadcasted_iota(jnp.int32, vals.shape, 1)
    ti = _extract16(vals, cid)
    row = lax.broadcasted_iota(jnp.int32, ti.shape, 0) + i * rc
    gid_ref[...] = row * nt + ti


def _chunk_select(TM, nt):
    B = TM.shape[0]
    rc = min(256, B)
    return pl.pallas_call(
        functools.partial(_chunksel_body, nt, rc),
        grid=(B // rc,),
        in_specs=[pl.BlockSpec((rc, nt), lambda i: (i, 0))],
        out_specs=pl.BlockSpec((rc, KNN), lambda i: (i, 0)),
        out_shape=jax.ShapeDtypeStruct((B, KNN), jnp.int32),
    )(TM)


def _extract16(vals, ids):
    out_i = []
    big = jnp.int32(2**31 - 1)
    for _ in range(KNN):
        m = jnp.max(vals, axis=1, keepdims=True)
        sel = jnp.where(vals == m, ids, big)
        win = jnp.min(sel, axis=1, keepdims=True)
        out_i.append(win)
        vals = jnp.where(ids == win, -jnp.inf, vals)
    return jnp.concatenate(out_i, axis=1)


def _finalsel_body(nt, rc, cv_ref, gid_ref, idx_ref):
    i = pl.program_id(0)
    vals = cv_ref[...]
    gid = gid_ref[...]
    row = lax.broadcasted_iota(jnp.int32, gid.shape, 0) + i * rc
    base = (gid - row * nt) * CW
    lane = lax.broadcasted_iota(jnp.int32, (rc, CW), 1)
    ids = jnp.concatenate(
        [base[:, j:j + 1] + lane for j in range(KNN)], axis=1)
    idx_ref[...] = _extract16(vals, ids)


def _final_select(cand, GID, nt):
    B = GID.shape[0]
    rc = min(256, B)
    return pl.pallas_call(
        functools.partial(_finalsel_body, nt, rc),
        grid=(B // rc,),
        in_specs=[
            pl.BlockSpec((rc, KNN * CW), lambda i: (i, 0)),
            pl.BlockSpec((rc, KNN), lambda i: (i, 0)),
        ],
        out_specs=pl.BlockSpec((rc, KNN), lambda i: (i, 0)),
        out_shape=jax.ShapeDtypeStruct((B, KNN), jnp.int32),
    )(cand, GID)


def _sc_gather_chunks(S2, gid_flat):
    nrow = gid_flat.shape[0]
    per_w = nrow // SC_NW
    ch = 128
    mesh = plsc.VectorSubcoreMesh(core_axis_name="c", subcore_axis_name="s")

    @functools.partial(
        pl.kernel, mesh=mesh,
        out_type=jax.ShapeDtypeStruct((nrow, CW), jnp.float32),
        scratch_types=[
            pltpu.VMEM((ch,), jnp.int32),
            pltpu.VMEM((ch, CW), jnp.float32),
            pltpu.SemaphoreType.DMA,
        ],
    )
    def k(tbl, idxh, outh, idx_v, rows_v, sem):
        wid = lax.axis_index("s") * SC_NC + lax.axis_index("c")
        base = wid * per_w
        for c in range(per_w // ch):
            pltpu.sync_copy(idxh.at[pl.ds(base + c * ch, ch)], idx_v)
            pltpu.async_copy(tbl.at[idx_v], rows_v, sem).wait()
            pltpu.sync_copy(rows_v, outh.at[pl.ds(base + c * ch, ch)])

    return k(S2, gid_flat)


def _sc_gather_rows(mem_keys, mem_vals, idx_flat):
    nrow = idx_flat.shape[0]
    D = mem_keys.shape[1]
    per_w = nrow // SC_NW
    ch = 64
    mesh = plsc.VectorSubcoreMesh(core_axis_name="c", subcore_axis_name="s")

    @functools.partial(
        pl.kernel, mesh=mesh,
        out_type=(jax.ShapeDtypeStruct((nrow, D), jnp.float32),
                  jax.ShapeDtypeStruct((nrow, D), jnp.float32)),
        scratch_types=[
            pltpu.VMEM((per_w,), jnp.int32),
            pltpu.VMEM((ch, D), jnp.float32),
            pltpu.VMEM((ch, D), jnp.float32),
            pltpu.SemaphoreType.DMA,
            pltpu.SemaphoreType.DMA,
        ],
    )
    def k(keys_h, vals_h, idxh, kout, vout, idx_v, krows, vrows, ksem, vsem):
        wid = lax.axis_index("s") * SC_NC + lax.axis_index("c")
        base = wid * per_w
        pltpu.sync_copy(idxh.at[pl.ds(base, per_w)], idx_v)
        for c in range(per_w // ch):
            isl = idx_v.at[pl.ds(c * ch, ch)]
            kcp = pltpu.async_copy(keys_h.at[isl], krows, ksem)
            vcp = pltpu.async_copy(vals_h.at[isl], vrows, vsem)
            kcp.wait()
            pltpu.sync_copy(krows, kout.at[pl.ds(base + c * ch, ch)])
            vcp.wait()
            pltpu.sync_copy(vrows, vout.at[pl.ds(base + c * ch, ch)])

    return k(mem_keys, mem_vals, idx_flat)


def _tail_body(rt, q_ref, kn_ref, vn_ref, wqc_ref, wkc_ref, wvc_ref,
               bc_ref, wo_ref, bo_ref, w1_ref, b1_ref, w2_ref,
               b2_ref, w3_ref, b3_ref, out_ref):
    D = q_ref.shape[1]
    dh = D // H
    Qb = q_ref[...]

    q2 = _dot_t(Qb, wqc_ref[...], TAIL_PREC) + bc_ref[0:1, :]
    Kn = kn_ref[...]
    Vn = vn_ref[...]

    lgs = []
    for h in range(H):
        sl = slice(h * dh, (h + 1) * dh)
        gh = _dot(q2[:, sl], wkc_ref[sl, :], TAIL_PREC)
        bh = jnp.sum(q2[:, sl] * bc_ref[1:2, sl], axis=1)
        lh = jnp.sum(Kn * gh[None, :, :], axis=2, keepdims=True)
        lgs.append(lh + bh[None, :, None])
    logits = jnp.concatenate(lgs, axis=2)
    logits = logits * (1.0 / math.sqrt(dh))
    mx = jnp.max(logits, axis=0, keepdims=True)
    p = jnp.exp(logits - mx)
    p = p / jnp.sum(p, axis=0, keepdims=True)
    pb = jnp.concatenate(
        [jnp.broadcast_to(p[:, :, h:h + 1], (KNN, rt, dh)) for h in range(H)],
        axis=2)
    vsum = jnp.sum(pb * Vn, axis=0)
    ctx = _dot_t(vsum, wvc_ref[...], TAIL_PREC) + bc_ref[2:3, :]

    prior = _dot_t(ctx, wo_ref[...], TAIL_PREC) + bo_ref[...]
    feat1 = _dot_t(Qb, w1_ref[...], TAIL_PREC) + b1_ref[...]
    feat2 = _dot_t(prior, w2_ref[...], TAIL_PREC) + b2_ref[...]
    feat = jnp.concatenate([feat1, feat2], axis=1)
    feat = feat * 0.5 * (1.0 + lax.erf(feat * (1.0 / math.sqrt(2.0))))
    out_ref[...] = _dot_t(feat, w3_ref[...], TAIL_PREC) + b3_ref[...]


def _tail(Q, KnT, VnT, Wqc, Wkc, Wvc, bc, Wo, bo, W1, b1, W2, b2, W3, b3):
    B, D = Q.shape
    rt = min(256, B)
    o3 = W3.shape[0]
    full = lambda a: pl.BlockSpec(a.shape, lambda i: (0,) * a.ndim)
    return pl.pallas_call(
        functools.partial(_tail_body, rt),
        grid=(B // rt,),
        in_specs=[
            pl.BlockSpec((rt, D), lambda i: (i, 0)),
            pl.BlockSpec((KNN, rt, D), lambda i: (0, i, 0)),
            pl.BlockSpec((KNN, rt, D), lambda i: (0, i, 0)),
            full(Wqc), full(Wkc), full(Wvc), full(bc), full(Wo), full(bo),
            full(W1), full(b1), full(W2), full(b2), full(W3), full(b3),
        ],
        out_specs=pl.BlockSpec((rt, o3), lambda i: (i, 0)),
        out_shape=jax.ShapeDtypeStruct((B, o3), jnp.float32),
        compiler_params=_BIG_VMEM,
    )(Q, KnT, VnT, Wqc, Wkc, Wvc, bc, Wo, bo, W1, b1, W2, b2, W3, b3)


def kernel(x, mem_keys, mem_vals, W_img, b_img, Wq, bq, Wk, bk, Wv, bv,
           W_in, b_in, Wo, bo, W1, b1, W2, b2, W3, b3):
    B, D = x.shape
    M = mem_keys.shape[0]

    b_pre = jnp.stack([bq, bk, bv], axis=0)
    b_in3 = b_in.reshape(3, D)
    Q, Wqc, Wkc, Wvc, bc = _project(x, W_img, b_img, Wq, Wk, Wv, W_in,
                                    b_pre, b_in3)
    S, TM3, nt = _distances(Q, mem_keys)
    TM = TM3.transpose(1, 0, 2).reshape(B, nt)
    GID = _chunk_select(TM, nt)
    cand = _sc_gather_chunks(S.reshape(B * nt, CW), GID.reshape(-1))
    idx = _final_select(cand.reshape(B, KNN * CW), GID, nt)

    idx_t = idx.T.reshape(-1)
    KnF, VnF = _sc_gather_rows(mem_keys, mem_vals, idx_t)
    KnT = KnF.reshape(KNN, B, D)
    VnT = VnF.reshape(KNN, B, D)

    feat = _tail(Q, KnT, VnT, Wqc, Wkc, Wvc, bc, Wo,
                 bo.reshape(1, D), W1, b1.reshape(1, -1), W2,
                 b2.reshape(1, -1), W3, b3.reshape(1, -1))
    return feat, Q


# speedup vs baseline: 8.9587x; 1.0172x over previous
import functools
import math

import jax
import jax.numpy as jnp
from jax import lax
from jax.experimental import pallas as pl
from jax.experimental.pallas import tpu as pltpu
from jax.experimental.pallas import tpu_sc as plsc

KNN = 16
H = 8
CW = 128
TK = 3072
_BIG_VMEM = pltpu.CompilerParams(vmem_limit_bytes=100 * 1024 * 1024)
NEG = -1e30

SC_NC = 2
SC_NS = 16
SC_NW = SC_NC * SC_NS

DIST_PREC = lax.Precision.DEFAULT
TAIL_PREC = lax.Precision.DEFAULT


def _dot_t(a, b, prec):
    return lax.dot_general(a, b, (((1,), (1,)), ((), ())), precision=prec,
                           preferred_element_type=jnp.float32)


def _dot(a, b, prec):
    return lax.dot_general(a, b, (((1,), (0,)), ((), ())), precision=prec,
                           preferred_element_type=jnp.float32)


def _dist_body(M, x_ref, wimg_ref, bimg_ref, k_ref, s_ref, tm_ref, q_ref):
    j = pl.program_id(0)

    @pl.when(j == 0)
    def _proj():
        q_ref[...] = (_dot_t(x_ref[...], wimg_ref[...], DIST_PREC)
                      + bimg_ref[...])

    Qb = q_ref[...]
    Kb = k_ref[...]
    B = Qb.shape[0]
    nc = TK // CW
    kvalid = (lax.broadcasted_iota(jnp.int32, (TK, 1), 0) + j * TK) < M
    Kb = jnp.where(kvalid, Kb, 0.0)
    qsq = jnp.sum(Qb * Qb, axis=1, keepdims=True)
    ksq = jnp.sum(Kb * Kb, axis=1, keepdims=True)
    ksq = jnp.where(kvalid, ksq, -NEG)
    qk = _dot_t(Qb, Kb, DIST_PREC)
    s = -(qsq - 2.0 * qk + ksq[:, 0][None, :])
    s_ref[...] = s.reshape(B, nc, CW)
    tm_ref[0] = jnp.max(s.reshape(B, nc, CW), axis=2)


def _distances(x, W_img, b_img, mem_keys):
    B, D = x.shape
    M = mem_keys.shape[0]
    nstep = math.ceil(M / TK)
    nt = nstep * (TK // CW)
    full = lambda a: pl.BlockSpec(a.shape, lambda j: (0,) * a.ndim)
    S, TM3, Q = pl.pallas_call(
        functools.partial(_dist_body, M),
        grid=(nstep,),
        in_specs=[
            full(x), full(W_img), pl.BlockSpec((1, D), lambda j: (0, 0)),
            pl.BlockSpec((TK, D), lambda j: (j, 0)),
        ],
        out_specs=[
            pl.BlockSpec((B, TK // CW, CW), lambda j: (0, j, 0)),
            pl.BlockSpec((1, B, TK // CW), lambda j: (j, 0, 0)),
            pl.BlockSpec((B, D), lambda j: (0, 0)),
        ],
        out_shape=[
            jax.ShapeDtypeStruct((B, nt, CW), jnp.float32),
            jax.ShapeDtypeStruct((nstep, B, TK // CW), jnp.float32),
            jax.ShapeDtypeStruct((B, D), jnp.float32),
        ],
        compiler_params=_BIG_VMEM,
    )(x, W_img, b_img.reshape(1, D), mem_keys)
    return S, TM3, Q, nt


def _chunksel_body(nt, rc, tm_ref, q_ref, w1_ref, b1_ref, wq_ref, wk_ref,
                   wv_ref, win_ref, bpre_ref, bin_ref,
                   gid_ref, f1_ref, wqc_ref, wkc_ref, wvc_ref, bc_ref):
    i = pl.program_id(0)
    D = q_ref.shape[1]
    vals = tm_ref[...]
    cid = lax.broadcasted_iota(jnp.int32, vals.shape, 1)
    ti = _extract16(vals, cid)
    row = lax.broadcasted_iota(jnp.int32, ti.shape, 0) + i * rc
    gid_ref[...] = row * nt + ti
    f1_ref[...] = _dot_t(q_ref[...], w1_ref[...], TAIL_PREC) + b1_ref[...]

    @pl.when(i == 0)
    def _weights():
        wiq = win_ref[0:D, :]
        wik = win_ref[D:2 * D, :]
        wiv = win_ref[2 * D:3 * D, :]
        wqc_ref[...] = _dot(wiq, wq_ref[...], TAIL_PREC)
        wkc_ref[...] = _dot(wik, wk_ref[...], TAIL_PREC)
        wvc_ref[...] = _dot(wiv, wv_ref[...], TAIL_PREC)
        bc_ref[...] = jnp.concatenate(
            [_dot_t(bpre_ref[0:1, :], wiq, TAIL_PREC) + bin_ref[0:1, :],
             _dot_t(bpre_ref[1:2, :], wik, TAIL_PREC) + bin_ref[1:2, :],
             _dot_t(bpre_ref[2:3, :], wiv, TAIL_PREC) + bin_ref[2:3, :]],
            axis=0)


def _chunk_select(TM, Q, W1, b1, Wq, Wk, Wv, W_in, b_pre, b_in3, nt):
    B, D = Q.shape
    o1 = W1.shape[0]
    rc = min(256, B)
    f32 = jnp.float32
    full = lambda a: pl.BlockSpec(a.shape, lambda i: (0,) * a.ndim)
    return pl.pallas_call(
        functools.partial(_chunksel_body, nt, rc),
        grid=(B // rc,),
        in_specs=[
            pl.BlockSpec((rc, nt), lambda i: (i, 0)),
            pl.BlockSpec((rc, D), lambda i: (i, 0)),
            full(W1), full(b1), full(Wq), full(Wk), full(Wv), full(W_in),
            full(b_pre), full(b_in3),
        ],
        out_specs=[
            pl.BlockSpec((rc, KNN), lambda i: (i, 0)),
            pl.BlockSpec((rc, o1), lambda i: (i, 0)),
            pl.BlockSpec((D, D), lambda i: (0, 0)),
            pl.BlockSpec((D, D), lambda i: (0, 0)),
            pl.BlockSpec((D, D), lambda i: (0, 0)),
            pl.BlockSpec((3, D), lambda i: (0, 0)),
        ],
        out_shape=[
            jax.ShapeDtypeStruct((B, KNN), jnp.int32),
            jax.ShapeDtypeStruct((B, o1), f32),
            jax.ShapeDtypeStruct((D, D), f32),
            jax.ShapeDtypeStruct((D, D), f32),
            jax.ShapeDtypeStruct((D, D), f32),
            jax.ShapeDtypeStruct((3, D), f32),
        ],
    )(TM, Q, W1, b1, Wq, Wk, Wv, W_in, b_pre, b_in3)


def _extract16(vals, ids):
    out_i = []
    big = jnp.int32(2**31 - 1)
    for _ in range(KNN):
        m = jnp.max(vals, axis=1, keepdims=True)
        sel = jnp.where(vals == m, ids, big)
        win = jnp.min(sel, axis=1, keepdims=True)
        out_i.append(win)
        vals = jnp.where(ids == win, -jnp.inf, vals)
    return jnp.concatenate(out_i, axis=1)


def _finalsel_body(nt, rc, cv_ref, gid_ref, idx_ref):
    i = pl.program_id(0)
    vals = cv_ref[...]
    gid = gid_ref[...]
    row = lax.broadcasted_iota(jnp.int32, gid.shape, 0) + i * rc
    base = (gid - row * nt) * CW
    lane = lax.broadcasted_iota(jnp.int32, (rc, CW), 1)
    ids = jnp.concatenate(
        [base[:, j:j + 1] + lane for j in range(KNN)], axis=1)
    idx_ref[...] = _extract16(vals, ids)


def _final_select(cand, GID, nt):
    B = GID.shape[0]
    rc = min(256, B)
    return pl.pallas_call(
        functools.partial(_finalsel_body, nt, rc),
        grid=(B // rc,),
        in_specs=[
            pl.BlockSpec((rc, KNN * CW), lambda i: (i, 0)),
            pl.BlockSpec((rc, KNN), lambda i: (i, 0)),
        ],
        out_specs=pl.BlockSpec((rc, KNN), lambda i: (i, 0)),
        out_shape=jax.ShapeDtypeStruct((B, KNN), jnp.int32),
    )(cand, GID)


def _sc_gather_chunks(S2, gid_flat):
    nrow = gid_flat.shape[0]
    per_w = nrow // SC_NW
    ch = 128
    mesh = plsc.VectorSubcoreMesh(core_axis_name="c", subcore_axis_name="s")

    @functools.partial(
        pl.kernel, mesh=mesh,
        out_type=jax.ShapeDtypeStruct((nrow, CW), jnp.float32),
        scratch_types=[
            pltpu.VMEM((ch,), jnp.int32),
            pltpu.VMEM((ch, CW), jnp.float32),
            pltpu.SemaphoreType.DMA,
        ],
    )
    def k(tbl, idxh, outh, idx_v, rows_v, sem):
        wid = lax.axis_index("s") * SC_NC + lax.axis_index("c")
        base = wid * per_w
        for c in range(per_w // ch):
            pltpu.sync_copy(idxh.at[pl.ds(base + c * ch, ch)], idx_v)
            pltpu.async_copy(tbl.at[idx_v], rows_v, sem).wait()
            pltpu.sync_copy(rows_v, outh.at[pl.ds(base + c * ch, ch)])

    return k(S2, gid_flat)


def _sc_gather_rows(mem_keys, mem_vals, idx_flat):
    nrow = idx_flat.shape[0]
    D = mem_keys.shape[1]
    per_w = nrow // SC_NW
    ch = 64
    mesh = plsc.VectorSubcoreMesh(core_axis_name="c", subcore_axis_name="s")

    @functools.partial(
        pl.kernel, mesh=mesh,
        out_type=(jax.ShapeDtypeStruct((nrow, D), jnp.float32),
                  jax.ShapeDtypeStruct((nrow, D), jnp.float32)),
        scratch_types=[
            pltpu.VMEM((per_w,), jnp.int32),
            pltpu.VMEM((ch, D), jnp.float32),
            pltpu.VMEM((ch, D), jnp.float32),
            pltpu.SemaphoreType.DMA,
            pltpu.SemaphoreType.DMA,
        ],
    )
    def k(keys_h, vals_h, idxh, kout, vout, idx_v, krows, vrows, ksem, vsem):
        wid = lax.axis_index("s") * SC_NC + lax.axis_index("c")
        base = wid * per_w
        pltpu.sync_copy(idxh.at[pl.ds(base, per_w)], idx_v)
        for c in range(per_w // ch):
            isl = idx_v.at[pl.ds(c * ch, ch)]
            kcp = pltpu.async_copy(keys_h.at[isl], krows, ksem)
            vcp = pltpu.async_copy(vals_h.at[isl], vrows, vsem)
            kcp.wait()
            pltpu.sync_copy(krows, kout.at[pl.ds(base + c * ch, ch)])
            vcp.wait()
            pltpu.sync_copy(vrows, vout.at[pl.ds(base + c * ch, ch)])

    return k(mem_keys, mem_vals, idx_flat)


def _tail_body(rt, q_ref, f1_ref, kn_ref, vn_ref, wqc_ref, wkc_ref, wvc_ref,
               bc_ref, wo_ref, bo_ref, w2_ref, b2_ref, w3_ref, b3_ref,
               out_ref):
    D = q_ref.shape[1]
    dh = D // H
    Qb = q_ref[...]

    q2 = _dot_t(Qb, wqc_ref[...], TAIL_PREC) + bc_ref[0:1, :]
    Kn = kn_ref[...]
    Vn = vn_ref[...]

    lgs = []
    for h in range(H):
        sl = slice(h * dh, (h + 1) * dh)
        gh = _dot(q2[:, sl], wkc_ref[sl, :], TAIL_PREC)
        bh = jnp.sum(q2[:, sl] * bc_ref[1:2, sl], axis=1)
        lh = jnp.sum(Kn * gh[None, :, :], axis=2, keepdims=True)
        lgs.append(lh + bh[None, :, None])
    logits = jnp.concatenate(lgs, axis=2)
    logits = logits * (1.0 / math.sqrt(dh))
    mx = jnp.max(logits, axis=0, keepdims=True)
    p = jnp.exp(logits - mx)
    p = p / jnp.sum(p, axis=0, keepdims=True)
    pb = jnp.concatenate(
        [jnp.broadcast_to(p[:, :, h:h + 1], (KNN, rt, dh)) for h in range(H)],
        axis=2)
    vsum = jnp.sum(pb * Vn, axis=0)
    ctx = _dot_t(vsum, wvc_ref[...], TAIL_PREC) + bc_ref[2:3, :]

    prior = _dot_t(ctx, wo_ref[...], TAIL_PREC) + bo_ref[...]
    feat2 = _dot_t(prior, w2_ref[...], TAIL_PREC) + b2_ref[...]
    feat = jnp.concatenate([f1_ref[...], feat2], axis=1)
    feat = feat * 0.5 * (1.0 + lax.erf(feat * (1.0 / math.sqrt(2.0))))
    out_ref[...] = _dot_t(feat, w3_ref[...], TAIL_PREC) + b3_ref[...]


def _tail(Q, F1, KnT, VnT, Wqc, Wkc, Wvc, bc, Wo, bo, W2, b2, W3, b3):
    B, D = Q.shape
    rt = min(256, B)
    o1 = F1.shape[1]
    o3 = W3.shape[0]
    full = lambda a: pl.BlockSpec(a.shape, lambda i: (0,) * a.ndim)
    return pl.pallas_call(
        functools.partial(_tail_body, rt),
        grid=(B // rt,),
        in_specs=[
            pl.BlockSpec((rt, D), lambda i: (i, 0)),
            pl.BlockSpec((rt, o1), lambda i: (i, 0)),
            pl.BlockSpec((KNN, rt, D), lambda i: (0, i, 0)),
            pl.BlockSpec((KNN, rt, D), lambda i: (0, i, 0)),
            full(Wqc), full(Wkc), full(Wvc), full(bc), full(Wo), full(bo),
            full(W2), full(b2), full(W3), full(b3),
        ],
        out_specs=pl.BlockSpec((rt, o3), lambda i: (i, 0)),
        out_shape=jax.ShapeDtypeStruct((B, o3), jnp.float32),
        compiler_params=_BIG_VMEM,
    )(Q, F1, KnT, VnT, Wqc, Wkc, Wvc, bc, Wo, bo, W2, b2, W3, b3)


def kernel(x, mem_keys, mem_vals, W_img, b_img, Wq, bq, Wk, bk, Wv, bv,
           W_in, b_in, Wo, bo, W1, b1, W2, b2, W3, b3):
    B, D = x.shape
    M = mem_keys.shape[0]

    b_pre = jnp.stack([bq, bk, bv], axis=0)
    b_in3 = b_in.reshape(3, D)
    S, TM3, Q, nt = _distances(x, W_img, b_img, mem_keys)
    TM = TM3.transpose(1, 0, 2).reshape(B, nt)
    GID, F1, Wqc, Wkc, Wvc, bc = _chunk_select(
        TM, Q, W1, b1.reshape(1, -1), Wq, Wk, Wv, W_in, b_pre, b_in3, nt)
    cand = _sc_gather_chunks(S.reshape(B * nt, CW), GID.reshape(-1))
    idx = _final_select(cand.reshape(B, KNN * CW), GID, nt)

    idx_t = idx.T.reshape(-1)
    KnF, VnF = _sc_gather_rows(mem_keys, mem_vals, idx_t)
    KnT = KnF.reshape(KNN, B, D)
    VnT = VnF.reshape(KNN, B, D)

    feat = _tail(Q, F1, KnT, VnT, Wqc, Wkc, Wvc, bc, Wo,
                 bo.reshape(1, D), W2, b2.reshape(1, -1), W3,
                 b3.reshape(1, -1))
    return feat, Q
